# Initial kernel scaffold; baseline (speedup 1.0000x reference)
import functools

import jax
import jax.numpy as jnp
import numpy as np
from jax import lax
from jax.experimental import pallas as pl
from jax.experimental.pallas import tpu as pltpu
from jax.experimental.pallas import tpu_sc as plsc

N = 10000
DIM = 128
H = 4
DH = 32
HALF = 64
NP = 10240
NTILES = 16
NCORES = 2
EB = 128
ROWS_PER_TILE = NP // NTILES


def _pre_body(x_ref, wg_ref, a8_ref, wsr_ref, h_ref, a8o_ref, xr_ref):
    xb = x_ref[...]
    hb = jnp.dot(xb, wg_ref[...], preferred_element_type=jnp.float32)
    h_ref[...] = hb
    a8o_ref[...] = jnp.dot(hb, a8_ref[...], preferred_element_type=jnp.float32)
    xr_ref[...] = jnp.dot(xb, wsr_ref[...], preferred_element_type=jnp.float32)


def _pre_call(xp, Wg, A8, Wsr):
    BR = 512
    full = pl.BlockSpec((DIM, DIM), lambda i: (0, 0))
    row = pl.BlockSpec((BR, DIM), lambda i: (i, 0))
    return pl.pallas_call(
        _pre_body,
        grid=(NP // BR,),
        in_specs=[row, full, full, full],
        out_specs=[row, row, row],
        out_shape=[jax.ShapeDtypeStruct((NP, DIM), jnp.float32)] * 3,
    )(xp, Wg, A8, Wsr)


def _sc_call(esrc, edst, th, tx, atab, z64, z16):
    E_pad = esrc.shape[0]
    EPT = E_pad // NTILES
    NBATCH = EPT // EB
    mesh = plsc.VectorSubcoreMesh(
        core_axis_name="c", subcore_axis_name="s",
        num_cores=NCORES, num_subcores=NTILES)

    @functools.partial(
        pl.kernel,
        out_type=[
            jax.ShapeDtypeStruct((NCORES, NP, HALF), jnp.float32),
            jax.ShapeDtypeStruct((NCORES, NP, HALF), jnp.float32),
            jax.ShapeDtypeStruct((NCORES, NP, 16), jnp.float32),
        ],
        mesh=mesh,
        scratch_types=[
            pltpu.VMEM((NP * 4,), jnp.float32),
            pltpu.VMEM((EB,), jnp.int32),
            pltpu.VMEM((EB,), jnp.int32),
            pltpu.VMEM((EB,), jnp.int32),
            pltpu.VMEM((EB, HALF), jnp.float32),
            pltpu.VMEM((EB, HALF), jnp.float32),
            pltpu.VMEM((EB, HALF), jnp.float32),
            pltpu.VMEM((EB, 16), jnp.float32),
            pltpu.VMEM_SHARED((NP, HALF), jnp.float32),
            pltpu.VMEM_SHARED((NP, HALF), jnp.float32),
            pltpu.VMEM_SHARED((NP, 16), jnp.float32),
            pltpu.SemaphoreType.DMA,
            pltpu.SemaphoreType.DMA,
        ],
    )
    def sc_kernel(esrc_r, edst_r, th_r, tx_r, atab_r, z64_r, z16_r,
                  msg_o, nsum_o, dcnt_o,
                  atab_v, srcv, gsrcv, dstv, hbuf, xbuf, wmsg, dcntb,
                  MSG, NSUM, DCNT, sem1, sem2):
        c = lax.axis_index("c")
        s = lax.axis_index("s")
        coff = c * NP
        pltpu.sync_copy(atab_r.at[c], atab_v)
        base_rows = s * ROWS_PER_TILE
        for i in range(ROWS_PER_TILE // EB):
            pltpu.sync_copy(z64_r, MSG.at[pl.ds(base_rows + i * EB, EB)])
            pltpu.sync_copy(z64_r, NSUM.at[pl.ds(base_rows + i * EB, EB)])
            pltpu.sync_copy(z16_r, DCNT.at[pl.ds(base_rows + i * EB, EB)])
        plsc.subcore_barrier()

        iota16 = lax.iota(jnp.int32, 16)
        col0 = jnp.zeros((16,), jnp.int32)
        col1 = jnp.full((16,), 1, jnp.int32)
        col2 = jnp.full((16,), 2, jnp.int32)
        ones_f = jnp.full((16,), 1.0, jnp.float32)

        def batch(i, carry):
            base = s * EPT + i * EB
            pltpu.sync_copy(esrc_r.at[pl.ds(base, EB)], srcv)
            pltpu.sync_copy(edst_r.at[pl.ds(base, EB)], dstv)
            for k in range(EB // 16):
                sv = srcv[pl.ds(k * 16, 16)]
                gsrcv[pl.ds(k * 16, 16)] = sv + coff
            g1 = pltpu.async_copy(th_r.at[gsrcv], hbuf, sem1)
            g2 = pltpu.async_copy(tx_r.at[gsrcv], xbuf, sem2)
            g2.wait()
            pltpu.sync_copy(xbuf, NSUM.at[dstv], add=True)
            g1.wait()
            for k in range(EB // 16):
                sv = srcv[pl.ds(k * 16, 16)]
                dv = dstv[pl.ds(k * 16, 16)]
                si = sv * 4
                di = dv * 4
                as0 = plsc.load_gather(atab_v, [si])
                as1 = plsc.load_gather(atab_v, [si + 1])
                ad0 = plsc.load_gather(atab_v, [di + 2])
                ad1 = plsc.load_gather(atab_v, [di + 3])
                z0 = as0 + ad0
                z1 = as1 + ad1
                w0 = jnp.exp(jnp.maximum(z0, 0.2 * z0))
                w1 = jnp.exp(jnp.maximum(z1, 0.2 * z1))
                rows = iota16 + (k * 16)
                plsc.store_scatter(dcntb, [rows, col0], w0)
                plsc.store_scatter(dcntb, [rows, col1], w1)
                plsc.store_scatter(dcntb, [rows, col2], ones_f)
                for j in range(16):
                    e = k * 16 + j
                    jidx = jnp.full((16,), j, jnp.int32)
                    b0 = jnp.take(w0, jidx, axis=0, mode="promise_in_bounds")
                    b1 = jnp.take(w1, jidx, axis=0, mode="promise_in_bounds")
                    wmsg[e, pl.ds(0, 16)] = hbuf[e, pl.ds(0, 16)] * b0
                    wmsg[e, pl.ds(16, 16)] = hbuf[e, pl.ds(16, 16)] * b0
                    wmsg[e, pl.ds(32, 16)] = hbuf[e, pl.ds(32, 16)] * b1
                    wmsg[e, pl.ds(48, 16)] = hbuf[e, pl.ds(48, 16)] * b1
            pltpu.sync_copy(wmsg, MSG.at[dstv], add=True)
            pltpu.sync_copy(dcntb, DCNT.at[dstv], add=True)
            return carry

        lax.fori_loop(0, NBATCH, batch, 0)
        plsc.subcore_barrier()
        pltpu.sync_copy(MSG.at[pl.ds(base_rows, ROWS_PER_TILE)],
                        msg_o.at[c, pl.ds(base_rows, ROWS_PER_TILE)])
        pltpu.sync_copy(NSUM.at[pl.ds(base_rows, ROWS_PER_TILE)],
                        nsum_o.at[c, pl.ds(base_rows, ROWS_PER_TILE)])
        pltpu.sync_copy(DCNT.at[pl.ds(base_rows, ROWS_PER_TILE)],
                        dcnt_o.at[c, pl.ds(base_rows, ROWS_PER_TILE)])

    return sc_kernel(esrc, edst, th, tx, atab, z64, z16)


def _post_body(xp_ref, h_ref, a8_ref, xr_ref, msg_ref, nsum_ref, dc_ref,
               wsl_ref, wpt_ref, wpb_ref, ssrc_ref, sdst_ref, sden_ref,
               scnt_ref, bias_ref, out_ref):
    a8b = a8_ref[...]
    asx = jnp.dot(a8b, ssrc_ref[...], preferred_element_type=jnp.float32)
    adx = jnp.dot(a8b, sdst_ref[...], preferred_element_type=jnp.float32)
    z = asx + adx
    wl = jnp.exp(jnp.maximum(z, 0.2 * z))
    hb = h_ref[...]
    msg_t = msg_ref[...] + hb * wl
    dcb = dc_ref[...]
    den = jnp.dot(dcb, sden_ref[...], preferred_element_type=jnp.float32) + wl + 1e-16
    gat = msg_t / den + bias_ref[0:1, :]
    cnt = jnp.maximum(jnp.dot(dcb, scnt_ref[...], preferred_element_type=jnp.float32), 1.0)
    mean = nsum_ref[...] / cnt
    sage = jnp.dot(mean, wsl_ref[...], preferred_element_type=jnp.float32) \
        + bias_ref[1:2, :] + xr_ref[...]
    o = jnp.dot(gat, wpt_ref[...], preferred_element_type=jnp.float32) \
        + jnp.dot(sage, wpb_ref[...], preferred_element_type=jnp.float32) \
        + bias_ref[2:3, :] + xp_ref[...]
    mu = jnp.mean(o, axis=-1, keepdims=True)
    d_ = o - mu
    var = jnp.mean(d_ * d_, axis=-1, keepdims=True)
    out_ref[...] = bias_ref[3:4, :] * (d_ * lax.rsqrt(var + 1e-5)) + bias_ref[4:5, :]


def _post_call(xp, h, a8, xr, msg, nsum, dc, Wsl, Wpt, Wpb, Ssrc, Sdst, Sden,
               Scnt, bias):
    BR = 512
    row = pl.BlockSpec((BR, DIM), lambda i: (i, 0))
    row32 = pl.BlockSpec((BR, 32), lambda i: (i, 0))
    full = pl.BlockSpec((DIM, DIM), lambda i: (0, 0))
    full32 = pl.BlockSpec((32, DIM), lambda i: (0, 0))
    fullb = pl.BlockSpec((8, DIM), lambda i: (0, 0))
    return pl.pallas_call(
        _post_body,
        grid=(NP // BR,),
        in_specs=[row, row, row, row, row, row, row32,
                  full, full, full, full, full, full32, full32, fullb],
        out_specs=row,
        out_shape=jax.ShapeDtypeStruct((NP, DIM), jnp.float32),
    )(xp, h, a8, xr, msg, nsum, dc, Wsl, Wpt, Wpb, Ssrc, Sdst, Sden, Scnt, bias)


def _selectors():
    ssrc = np.zeros((DIM, DIM), np.float32)
    sdst = np.zeros((DIM, DIM), np.float32)
    for hh in range(H):
        ssrc[hh, hh * DH:(hh + 1) * DH] = 1.0
        sdst[4 + hh, hh * DH:(hh + 1) * DH] = 1.0
    sden = np.zeros((32, DIM), np.float32)
    sden[0, 0:32] = 1.0
    sden[1, 32:64] = 1.0
    sden[16, 64:96] = 1.0
    sden[17, 96:128] = 1.0
    scnt = np.zeros((32, DIM), np.float32)
    scnt[2, :] = 1.0
    return (jnp.asarray(ssrc), jnp.asarray(sdst), jnp.asarray(sden),
            jnp.asarray(scnt))


_SSRC, _SDST, _SDEN, _SCNT = _selectors()


def kernel(x, edge_index, W_gat, att_src, att_dst, b_gat, W_sage_l, b_sage_l,
           W_sage_r, W_proj, b_proj, gamma, beta):
    E = edge_index.shape[1]
    E_pad = -(-E // (NTILES * EB)) * (NTILES * EB)

    xp = jnp.zeros((NP, DIM), jnp.float32).at[:N].set(x)

    A8 = jnp.zeros((DIM, DIM), jnp.float32)
    for hh in range(H):
        A8 = A8.at[hh * DH:(hh + 1) * DH, hh].set(att_src[hh])
        A8 = A8.at[hh * DH:(hh + 1) * DH, 4 + hh].set(att_dst[hh])

    h, a8, xr = _pre_call(xp, W_gat, A8, W_sage_r)

    pad = jnp.full((E_pad - E,), N, jnp.int32)
    esrc = jnp.concatenate([edge_index[0].astype(jnp.int32), pad])
    edst = jnp.concatenate([edge_index[1].astype(jnp.int32), pad])
    th = jnp.concatenate([h[:, :HALF], h[:, HALF:]], axis=0)
    tx = jnp.concatenate([xp[:, :HALF], xp[:, HALF:]], axis=0)
    atab = jnp.stack([
        jnp.stack([a8[:, 0], a8[:, 1], a8[:, 4], a8[:, 5]], axis=1).reshape(-1),
        jnp.stack([a8[:, 2], a8[:, 3], a8[:, 6], a8[:, 7]], axis=1).reshape(-1),
    ], axis=0)
    z64 = jnp.zeros((EB, HALF), jnp.float32)
    z16 = jnp.zeros((EB, 16), jnp.float32)

    msg2, nsum2, dcnt2 = _sc_call(esrc, edst, th, tx, atab, z64, z16)

    msg = jnp.concatenate([msg2[0], msg2[1]], axis=1)
    nsum = jnp.concatenate([nsum2[0], nsum2[1]], axis=1)
    dc = jnp.concatenate([dcnt2[0], dcnt2[1]], axis=1)

    bias = jnp.zeros((8, DIM), jnp.float32)
    bias = bias.at[0].set(b_gat).at[1].set(b_sage_l).at[2].set(b_proj)
    bias = bias.at[3].set(gamma).at[4].set(beta)

    out = _post_call(xp, h, a8, xr, msg, nsum, dc, W_sage_l,
                     W_proj[:DIM], W_proj[DIM:], _SSRC, _SDST, _SDEN, _SCNT,
                     bias)
    return out[:N]


# speedup vs baseline: 39.2077x; 39.2077x over previous
import functools

import jax
import jax.numpy as jnp
import numpy as np
from jax import lax
from jax.experimental import pallas as pl
from jax.experimental.pallas import tpu as pltpu
from jax.experimental.pallas import tpu_sc as plsc

N = 10000
DIM = 128
H = 4
DH = 32
HALF = 64
NP = 10240
NTILES = 16
NCORES = 2
EB = 128
ROWS_PER_TILE = NP // NTILES


def _pre_body(x_ref, wg_ref, a8_ref, wsr_ref, h_ref, a8o_ref, xr_ref):
    xb = x_ref[...]
    hb = jnp.dot(xb, wg_ref[...], preferred_element_type=jnp.float32)
    h_ref[...] = hb
    a8o_ref[...] = jnp.dot(hb, a8_ref[...], preferred_element_type=jnp.float32)
    xr_ref[...] = jnp.dot(xb, wsr_ref[...], preferred_element_type=jnp.float32)


def _pre_call(xp, Wg, A8, Wsr):
    BR = 512
    full = pl.BlockSpec((DIM, DIM), lambda i: (0, 0))
    row = pl.BlockSpec((BR, DIM), lambda i: (i, 0))
    return pl.pallas_call(
        _pre_body,
        grid=(NP // BR,),
        in_specs=[row, full, full, full],
        out_specs=[row, row, row],
        out_shape=[jax.ShapeDtypeStruct((NP, DIM), jnp.float32)] * 3,
    )(xp, Wg, A8, Wsr)


def _sc_mesh():
    return plsc.VectorSubcoreMesh(
        core_axis_name="c", subcore_axis_name="s",
        num_cores=NCORES, num_subcores=NTILES)


_SC_PARAMS = pltpu.CompilerParams(
    needs_layout_passes=False, use_tc_tiling_on_sc=False)


def _sc_gat_call(esrc, edst, th, atab, z64, z16):
    E_pad = esrc.shape[0]
    EPT = E_pad // NTILES
    NBATCH = EPT // EB

    @functools.partial(
        pl.kernel,
        out_type=[
            jax.ShapeDtypeStruct((NCORES, NP, HALF), jnp.float32),
            jax.ShapeDtypeStruct((NCORES, NP, 16), jnp.float32),
        ],
        mesh=_sc_mesh(),
        compiler_params=_SC_PARAMS,
        scratch_types=[
            pltpu.VMEM((NP * 4,), jnp.float32),
            pltpu.VMEM((EB,), jnp.int32),
            pltpu.VMEM((EB,), jnp.int32),
            pltpu.VMEM((EB,), jnp.int32),
            pltpu.VMEM((EB, HALF), jnp.float32),
            pltpu.VMEM((EB, HALF), jnp.float32),
            pltpu.VMEM((EB, 16), jnp.float32),
            pltpu.VMEM_SHARED((NP, HALF), jnp.float32),
            pltpu.VMEM_SHARED((NP, 16), jnp.float32),
            pltpu.SemaphoreType.DMA,
        ],
    )
    def sc_kernel(esrc_r, edst_r, th_r, atab_r, z64_r, z16_r,
                  msg_o, dcnt_o,
                  atab_v, srcv, gsrcv, dstv, hbuf, wmsg, dcntb,
                  MSG, DCNT, sem1):
        c = lax.axis_index("c")
        s = lax.axis_index("s")
        coff = c * NP
        pltpu.sync_copy(atab_r.at[c], atab_v)
        base_rows = s * ROWS_PER_TILE
        for i in range(ROWS_PER_TILE // EB):
            pltpu.sync_copy(z64_r, MSG.at[pl.ds(base_rows + i * EB, EB)])
            pltpu.sync_copy(z16_r, DCNT.at[pl.ds(base_rows + i * EB, EB)])
        plsc.subcore_barrier()

        gdn = lax.GatherDimensionNumbers(
            offset_dims=(), collapsed_slice_dims=(0,), start_index_map=(0,))

        def bcast_lane(vec, j):
            jidx = jnp.full((16, 1), j, jnp.int32)
            return lax.gather(
                vec, jidx, gdn, (1,),
                mode=lax.GatherScatterMode.PROMISE_IN_BOUNDS)

        iota16 = lax.iota(jnp.int32, 16)
        col0 = jnp.zeros((16,), jnp.int32)
        col1 = jnp.full((16,), 1, jnp.int32)
        col2 = jnp.full((16,), 2, jnp.int32)
        ones_f = jnp.full((16,), 1.0, jnp.float32)

        def batch(i, carry):
            base = s * EPT + i * EB
            pltpu.sync_copy(esrc_r.at[pl.ds(base, EB)], srcv)
            pltpu.sync_copy(edst_r.at[pl.ds(base, EB)], dstv)
            for k in range(EB // 16):
                sv = srcv[pl.ds(k * 16, 16)]
                gsrcv[pl.ds(k * 16, 16)] = sv + coff
            g1 = pltpu.async_copy(th_r.at[gsrcv], hbuf, sem1)
            g1.wait()
            for k in range(EB // 16):
                sv = srcv[pl.ds(k * 16, 16)]
                dv = dstv[pl.ds(k * 16, 16)]
                si = sv * 4
                di = dv * 4
                as0 = plsc.load_gather(atab_v, [si])
                as1 = plsc.load_gather(atab_v, [si + 1])
                ad0 = plsc.load_gather(atab_v, [di + 2])
                ad1 = plsc.load_gather(atab_v, [di + 3])
                z0 = as0 + ad0
                z1 = as1 + ad1
                w0 = jnp.exp(jnp.maximum(z0, 0.2 * z0))
                w1 = jnp.exp(jnp.maximum(z1, 0.2 * z1))
                rows = iota16 + (k * 16)
                plsc.store_scatter(dcntb, [rows, col0], w0)
                plsc.store_scatter(dcntb, [rows, col1], w1)
                plsc.store_scatter(dcntb, [rows, col2], ones_f)
                for j in range(16):
                    e = k * 16 + j
                    b0 = bcast_lane(w0, j)
                    b1 = bcast_lane(w1, j)
                    wmsg[e, pl.ds(0, 16)] = hbuf[e, pl.ds(0, 16)] * b0
                    wmsg[e, pl.ds(16, 16)] = hbuf[e, pl.ds(16, 16)] * b0
                    wmsg[e, pl.ds(32, 16)] = hbuf[e, pl.ds(32, 16)] * b1
                    wmsg[e, pl.ds(48, 16)] = hbuf[e, pl.ds(48, 16)] * b1
            pltpu.sync_copy(wmsg, MSG.at[dstv], add=True)
            pltpu.sync_copy(dcntb, DCNT.at[dstv], add=True)
            return carry

        lax.fori_loop(0, NBATCH, batch, 0)
        plsc.subcore_barrier()
        pltpu.sync_copy(MSG.at[pl.ds(base_rows, ROWS_PER_TILE)],
                        msg_o.at[c, pl.ds(base_rows, ROWS_PER_TILE)])
        pltpu.sync_copy(DCNT.at[pl.ds(base_rows, ROWS_PER_TILE)],
                        dcnt_o.at[c, pl.ds(base_rows, ROWS_PER_TILE)])

    return sc_kernel(esrc, edst, th, atab, z64, z16)


def _sc_sage_call(esrc, edst, tx, z64):
    E_pad = esrc.shape[0]
    EPT = E_pad // NTILES
    NBATCH = EPT // EB

    @functools.partial(
        pl.kernel,
        out_type=jax.ShapeDtypeStruct((NCORES, NP, HALF), jnp.float32),
        mesh=_sc_mesh(),
        compiler_params=_SC_PARAMS,
        scratch_types=[
            pltpu.VMEM((EB,), jnp.int32),
            pltpu.VMEM((EB,), jnp.int32),
            pltpu.VMEM((EB,), jnp.int32),
            pltpu.VMEM((EB, HALF), jnp.float32),
            pltpu.VMEM_SHARED((NP, HALF), jnp.float32),
            pltpu.SemaphoreType.DMA,
        ],
    )
    def sc_kernel(esrc_r, edst_r, tx_r, z64_r, nsum_o,
                  srcv, gsrcv, dstv, xbuf, NSUM, sem1):
        c = lax.axis_index("c")
        s = lax.axis_index("s")
        coff = c * NP
        base_rows = s * ROWS_PER_TILE
        for i in range(ROWS_PER_TILE // EB):
            pltpu.sync_copy(z64_r, NSUM.at[pl.ds(base_rows + i * EB, EB)])
        plsc.subcore_barrier()

        def batch(i, carry):
            base = s * EPT + i * EB
            pltpu.sync_copy(esrc_r.at[pl.ds(base, EB)], srcv)
            pltpu.sync_copy(edst_r.at[pl.ds(base, EB)], dstv)
            for k in range(EB // 16):
                sv = srcv[pl.ds(k * 16, 16)]
                gsrcv[pl.ds(k * 16, 16)] = sv + coff
            pltpu.async_copy(tx_r.at[gsrcv], xbuf, sem1).wait()
            pltpu.sync_copy(xbuf, NSUM.at[dstv], add=True)
            return carry

        lax.fori_loop(0, NBATCH, batch, 0)
        plsc.subcore_barrier()
        pltpu.sync_copy(NSUM.at[pl.ds(base_rows, ROWS_PER_TILE)],
                        nsum_o.at[c, pl.ds(base_rows, ROWS_PER_TILE)])

    return sc_kernel(esrc, edst, tx, z64)


def _post_body(xp_ref, h_ref, a8_ref, xr_ref, msg_ref, nsum_ref, dc_ref,
               wsl_ref, wpt_ref, wpb_ref, ssrc_ref, sdst_ref, sden_ref,
               scnt_ref, bias_ref, out_ref):
    a8b = a8_ref[...]
    asx = jnp.dot(a8b, ssrc_ref[...], preferred_element_type=jnp.float32)
    adx = jnp.dot(a8b, sdst_ref[...], preferred_element_type=jnp.float32)
    z = asx + adx
    wl = jnp.exp(jnp.maximum(z, 0.2 * z))
    hb = h_ref[...]
    msg_t = msg_ref[...] + hb * wl
    dcb = dc_ref[...]
    den = jnp.dot(dcb, sden_ref[...], preferred_element_type=jnp.float32) + wl + 1e-16
    gat = msg_t / den + bias_ref[0:1, :]
    cnt = jnp.maximum(jnp.dot(dcb, scnt_ref[...], preferred_element_type=jnp.float32), 1.0)
    mean = nsum_ref[...] / cnt
    sage = jnp.dot(mean, wsl_ref[...], preferred_element_type=jnp.float32) \
        + bias_ref[1:2, :] + xr_ref[...]
    o = jnp.dot(gat, wpt_ref[...], preferred_element_type=jnp.float32) \
        + jnp.dot(sage, wpb_ref[...], preferred_element_type=jnp.float32) \
        + bias_ref[2:3, :] + xp_ref[...]
    mu = jnp.mean(o, axis=-1, keepdims=True)
    d_ = o - mu
    var = jnp.mean(d_ * d_, axis=-1, keepdims=True)
    out_ref[...] = bias_ref[3:4, :] * (d_ * lax.rsqrt(var + 1e-5)) + bias_ref[4:5, :]


def _post_call(xp, h, a8, xr, msg, nsum, dc, Wsl, Wpt, Wpb, Ssrc, Sdst, Sden,
               Scnt, bias):
    BR = 512
    row = pl.BlockSpec((BR, DIM), lambda i: (i, 0))
    row32 = pl.BlockSpec((BR, 32), lambda i: (i, 0))
    full = pl.BlockSpec((DIM, DIM), lambda i: (0, 0))
    full32 = pl.BlockSpec((32, DIM), lambda i: (0, 0))
    fullb = pl.BlockSpec((8, DIM), lambda i: (0, 0))
    return pl.pallas_call(
        _post_body,
        grid=(NP // BR,),
        in_specs=[row, row, row, row, row, row, row32,
                  full, full, full, full, full, full32, full32, fullb],
        out_specs=row,
        out_shape=jax.ShapeDtypeStruct((NP, DIM), jnp.float32),
    )(xp, h, a8, xr, msg, nsum, dc, Wsl, Wpt, Wpb, Ssrc, Sdst, Sden, Scnt, bias)


def _selectors():
    ssrc = np.zeros((DIM, DIM), np.float32)
    sdst = np.zeros((DIM, DIM), np.float32)
    for hh in range(H):
        ssrc[hh, hh * DH:(hh + 1) * DH] = 1.0
        sdst[4 + hh, hh * DH:(hh + 1) * DH] = 1.0
    sden = np.zeros((32, DIM), np.float32)
    sden[0, 0:32] = 1.0
    sden[1, 32:64] = 1.0
    sden[16, 64:96] = 1.0
    sden[17, 96:128] = 1.0
    scnt = np.zeros((32, DIM), np.float32)
    scnt[2, :] = 1.0
    return ssrc, sdst, sden, scnt


_SSRC, _SDST, _SDEN, _SCNT = _selectors()


def kernel(x, edge_index, W_gat, att_src, att_dst, b_gat, W_sage_l, b_sage_l,
           W_sage_r, W_proj, b_proj, gamma, beta):
    E = edge_index.shape[1]
    E_pad = -(-E // (NTILES * EB)) * (NTILES * EB)

    xp = jnp.zeros((NP, DIM), jnp.float32).at[:N].set(x)

    A8 = jnp.zeros((DIM, DIM), jnp.float32)
    for hh in range(H):
        A8 = A8.at[hh * DH:(hh + 1) * DH, hh].set(att_src[hh])
        A8 = A8.at[hh * DH:(hh + 1) * DH, 4 + hh].set(att_dst[hh])

    h, a8, xr = _pre_call(xp, W_gat, A8, W_sage_r)

    pad = jnp.full((E_pad - E,), N, jnp.int32)
    esrc = jnp.concatenate([edge_index[0].astype(jnp.int32), pad])
    edst = jnp.concatenate([edge_index[1].astype(jnp.int32), pad])
    th = jnp.concatenate([h[:, :HALF], h[:, HALF:]], axis=0)
    tx = jnp.concatenate([xp[:, :HALF], xp[:, HALF:]], axis=0)
    atab = jnp.stack([
        jnp.stack([a8[:, 0], a8[:, 1], a8[:, 4], a8[:, 5]], axis=1).reshape(-1),
        jnp.stack([a8[:, 2], a8[:, 3], a8[:, 6], a8[:, 7]], axis=1).reshape(-1),
    ], axis=0)
    z64 = jnp.zeros((EB, HALF), jnp.float32)
    z16 = jnp.zeros((EB, 16), jnp.float32)

    msg2, dcnt2 = _sc_gat_call(esrc, edst, th, atab, z64, z16)
    nsum2 = _sc_sage_call(esrc, edst, tx, z64)

    msg = jnp.concatenate([msg2[0], msg2[1]], axis=1)
    nsum = jnp.concatenate([nsum2[0], nsum2[1]], axis=1)
    dc = jnp.concatenate([dcnt2[0], dcnt2[1]], axis=1)

    bias = jnp.zeros((8, DIM), jnp.float32)
    bias = bias.at[0].set(b_gat).at[1].set(b_sage_l).at[2].set(b_proj)
    bias = bias.at[3].set(gamma).at[4].set(beta)

    out = _post_call(xp, h, a8, xr, msg, nsum, dc, W_sage_l,
                     W_proj[:DIM], W_proj[DIM:], _SSRC, _SDST, _SDEN, _SCNT,
                     bias)
    return out[:N]


# speedup vs baseline: 41.5884x; 1.0607x over previous
import functools

import jax
import jax.numpy as jnp
import numpy as np
from jax import lax
from jax.experimental import pallas as pl
from jax.experimental.pallas import tpu as pltpu
from jax.experimental.pallas import tpu_sc as plsc

N = 10000
DIM = 128
H = 4
DH = 32
HALF = 64
NP = 10240
NTILES = 16
NCORES = 2
EB = 128
ROWS_PER_TILE = NP // NTILES


def _pre_body(x_ref, wg_ref, a8_ref, wsr_ref, h_ref, a8o_ref, xr_ref):
    xb = x_ref[...]
    hb = jnp.dot(xb, wg_ref[...], preferred_element_type=jnp.float32)
    h_ref[...] = hb
    a8o_ref[...] = jnp.dot(hb, a8_ref[...], preferred_element_type=jnp.float32)
    xr_ref[...] = jnp.dot(xb, wsr_ref[...], preferred_element_type=jnp.float32)


def _pre_call(xp, Wg, A8, Wsr):
    BR = 512
    full = pl.BlockSpec((DIM, DIM), lambda i: (0, 0))
    row = pl.BlockSpec((BR, DIM), lambda i: (i, 0))
    return pl.pallas_call(
        _pre_body,
        grid=(NP // BR,),
        in_specs=[row, full, full, full],
        out_specs=[row, row, row],
        out_shape=[jax.ShapeDtypeStruct((NP, DIM), jnp.float32)] * 3,
    )(xp, Wg, A8, Wsr)


def _sc_mesh():
    return plsc.VectorSubcoreMesh(
        core_axis_name="c", subcore_axis_name="s",
        num_cores=NCORES, num_subcores=NTILES)


_SC_PARAMS = pltpu.CompilerParams(
    needs_layout_passes=False, use_tc_tiling_on_sc=False)


def _sc_call(esrc, edst, th, tx, atab, z64, z16):
    E_pad = esrc.shape[0]
    EPT = E_pad // NTILES
    SUB = 1
    MACRO = SUB * EB
    NM = EPT // MACRO

    @functools.partial(
        pl.kernel,
        out_type=[
            jax.ShapeDtypeStruct((NCORES, NP, HALF), jnp.float32),
            jax.ShapeDtypeStruct((NCORES, NP, 16), jnp.float32),
            jax.ShapeDtypeStruct((NCORES, NP, HALF), jnp.float32),
        ],
        mesh=_sc_mesh(),
        compiler_params=_SC_PARAMS,
        scratch_types=[
            pltpu.VMEM((NP * 4,), jnp.float32),
            pltpu.VMEM((2 * SUB, EB), jnp.int32),
            pltpu.VMEM((2 * SUB, EB), jnp.int32),
            pltpu.VMEM((2 * SUB, EB), jnp.int32),
            pltpu.VMEM((2, MACRO, HALF), jnp.float32),
            pltpu.VMEM((MACRO, HALF), jnp.float32),
            pltpu.VMEM((MACRO, 16), jnp.float32),
            pltpu.VMEM_SHARED((NP, HALF), jnp.float32),
            pltpu.VMEM_SHARED((NP, 16), jnp.float32),
            pltpu.SemaphoreType.DMA,
            pltpu.SemaphoreType.DMA,
        ],
    )
    def sc_kernel(esrc_r, edst_r, th_r, tx_r, atab_r, z64_r, z16_r,
                  msg_o, dcnt_o, nsum_o,
                  atab_v, srcv, gsrcv, dstv, hbuf, wmsg, dcntb,
                  ACC, DCNT, gsem0, gsem1):
        c = lax.axis_index("c")
        s = lax.axis_index("s")
        coff = c * NP
        gsems = (gsem0, gsem1)
        pltpu.sync_copy(atab_r.at[c], atab_v)
        base_rows = s * ROWS_PER_TILE
        for i in range(ROWS_PER_TILE // EB):
            pltpu.sync_copy(z64_r, ACC.at[pl.ds(base_rows + i * EB, EB)])
            pltpu.sync_copy(z16_r, DCNT.at[pl.ds(base_rows + i * EB, EB)])
        plsc.subcore_barrier()

        gdn = lax.GatherDimensionNumbers(
            offset_dims=(), collapsed_slice_dims=(0,), start_index_map=(0,))

        def bcast_lane(vec, j):
            jidx = jnp.full((16, 1), j, jnp.int32)
            return lax.gather(
                vec, jidx, gdn, (1,),
                mode=lax.GatherScatterMode.PROMISE_IN_BOUNDS)

        iota16 = lax.iota(jnp.int32, 16)
        col0 = jnp.zeros((16,), jnp.int32)
        col1 = jnp.full((16,), 1, jnp.int32)
        col2 = jnp.full((16,), 2, jnp.int32)
        ones_f = jnp.full((16,), 1.0, jnp.float32)
        tile_base = s * EPT

        def make_fetch(table_r):
            def fetch(m, buf):
                for j in range(SUB):
                    base = tile_base + m * MACRO + j * EB
                    r = SUB * buf + j
                    pltpu.sync_copy(esrc_r.at[pl.ds(base, EB)], srcv.at[r])
                    pltpu.sync_copy(edst_r.at[pl.ds(base, EB)], dstv.at[r])
                    for q in range(EB // 16):
                        sl = pl.ds(q * 16, 16)
                        gsrcv[r, sl] = srcv[r, sl] + coff
                    pltpu.async_copy(table_r.at[gsrcv.at[r]],
                                     hbuf.at[buf, pl.ds(j * EB, EB)],
                                     gsems[buf])
            return fetch

        def edge_loop(table_r, do_macro):
            fetch = make_fetch(table_r)
            fetch(0, 0)

            def macro_step(i, carry):
                for b in range(2):
                    nxt = 2 * i + b + 1
                    if b == 0:
                        fetch(nxt, 1)
                    else:
                        @pl.when(i < NM // 2 - 1)
                        def _():
                            fetch(nxt, 0)
                    for j in range(SUB):
                        pltpu.make_async_copy(
                            table_r.at[gsrcv.at[SUB * b + j]],
                            hbuf.at[b, pl.ds(j * EB, EB)], gsems[b]).wait()
                    do_macro(b)
                return carry

            lax.fori_loop(0, NM // 2, macro_step, 0)

        def gat_macro(b):
            for k in range(MACRO // 16):
                r = SUB * b + k // 8
                sl = pl.ds((k % 8) * 16, 16)
                sv = srcv[r, sl]
                dv = dstv[r, sl]
                si = sv * 4
                di = dv * 4
                as0 = plsc.load_gather(atab_v, [si])
                as1 = plsc.load_gather(atab_v, [si + 1])
                ad0 = plsc.load_gather(atab_v, [di + 2])
                ad1 = plsc.load_gather(atab_v, [di + 3])
                z0 = as0 + ad0
                z1 = as1 + ad1
                w0 = jnp.exp(jnp.maximum(z0, 0.2 * z0))
                w1 = jnp.exp(jnp.maximum(z1, 0.2 * z1))
                rows = iota16 + (k * 16)
                plsc.store_scatter(dcntb, [rows, col0], w0)
                plsc.store_scatter(dcntb, [rows, col1], w1)
                plsc.store_scatter(dcntb, [rows, col2], ones_f)
                for j in range(16):
                    e = k * 16 + j
                    b0 = bcast_lane(w0, j)
                    b1 = bcast_lane(w1, j)
                    wmsg[e, pl.ds(0, 16)] = hbuf[b, e, pl.ds(0, 16)] * b0
                    wmsg[e, pl.ds(16, 16)] = hbuf[b, e, pl.ds(16, 16)] * b0
                    wmsg[e, pl.ds(32, 16)] = hbuf[b, e, pl.ds(32, 16)] * b1
                    wmsg[e, pl.ds(48, 16)] = hbuf[b, e, pl.ds(48, 16)] * b1
            for j in range(SUB):
                pltpu.sync_copy(wmsg.at[pl.ds(j * EB, EB)],
                                ACC.at[dstv.at[SUB * b + j]], add=True)
                pltpu.sync_copy(dcntb.at[pl.ds(j * EB, EB)],
                                DCNT.at[dstv.at[SUB * b + j]], add=True)

        def sage_macro(b):
            for j in range(SUB):
                pltpu.sync_copy(hbuf.at[b, pl.ds(j * EB, EB)],
                                ACC.at[dstv.at[SUB * b + j]], add=True)

        edge_loop(th_r, gat_macro)
        plsc.subcore_barrier()
        pltpu.sync_copy(ACC.at[pl.ds(base_rows, ROWS_PER_TILE)],
                        msg_o.at[c, pl.ds(base_rows, ROWS_PER_TILE)])
        pltpu.sync_copy(DCNT.at[pl.ds(base_rows, ROWS_PER_TILE)],
                        dcnt_o.at[c, pl.ds(base_rows, ROWS_PER_TILE)])
        for i in range(ROWS_PER_TILE // EB):
            pltpu.sync_copy(z64_r, ACC.at[pl.ds(base_rows + i * EB, EB)])
        plsc.subcore_barrier()

        edge_loop(tx_r, sage_macro)
        plsc.subcore_barrier()
        pltpu.sync_copy(ACC.at[pl.ds(base_rows, ROWS_PER_TILE)],
                        nsum_o.at[c, pl.ds(base_rows, ROWS_PER_TILE)])

    return sc_kernel(esrc, edst, th, tx, atab, z64, z16)


def _post_body(xp_ref, h_ref, a8_ref, xr_ref, msg_ref, nsum_ref, dc_ref,
               wsl_ref, wpt_ref, wpb_ref, ssrc_ref, sdst_ref, sden_ref,
               scnt_ref, bias_ref, out_ref):
    a8b = a8_ref[...]
    asx = jnp.dot(a8b, ssrc_ref[...], preferred_element_type=jnp.float32)
    adx = jnp.dot(a8b, sdst_ref[...], preferred_element_type=jnp.float32)
    z = asx + adx
    wl = jnp.exp(jnp.maximum(z, 0.2 * z))
    hb = h_ref[...]
    msg_t = msg_ref[...] + hb * wl
    dcb = dc_ref[...]
    den = jnp.dot(dcb, sden_ref[...], preferred_element_type=jnp.float32) + wl + 1e-16
    gat = msg_t / den + bias_ref[0:1, :]
    cnt = jnp.maximum(jnp.dot(dcb, scnt_ref[...], preferred_element_type=jnp.float32), 1.0)
    mean = nsum_ref[...] / cnt
    sage = jnp.dot(mean, wsl_ref[...], preferred_element_type=jnp.float32) \
        + bias_ref[1:2, :] + xr_ref[...]
    o = jnp.dot(gat, wpt_ref[...], preferred_element_type=jnp.float32) \
        + jnp.dot(sage, wpb_ref[...], preferred_element_type=jnp.float32) \
        + bias_ref[2:3, :] + xp_ref[...]
    mu = jnp.mean(o, axis=-1, keepdims=True)
    d_ = o - mu
    var = jnp.mean(d_ * d_, axis=-1, keepdims=True)
    out_ref[...] = bias_ref[3:4, :] * (d_ * lax.rsqrt(var + 1e-5)) + bias_ref[4:5, :]


def _post_call(xp, h, a8, xr, msg, nsum, dc, Wsl, Wpt, Wpb, Ssrc, Sdst, Sden,
               Scnt, bias):
    BR = 512
    row = pl.BlockSpec((BR, DIM), lambda i: (i, 0))
    row32 = pl.BlockSpec((BR, 32), lambda i: (i, 0))
    full = pl.BlockSpec((DIM, DIM), lambda i: (0, 0))
    full32 = pl.BlockSpec((32, DIM), lambda i: (0, 0))
    fullb = pl.BlockSpec((8, DIM), lambda i: (0, 0))
    return pl.pallas_call(
        _post_body,
        grid=(NP // BR,),
        in_specs=[row, row, row, row, row, row, row32,
                  full, full, full, full, full, full32, full32, fullb],
        out_specs=row,
        out_shape=jax.ShapeDtypeStruct((NP, DIM), jnp.float32),
    )(xp, h, a8, xr, msg, nsum, dc, Wsl, Wpt, Wpb, Ssrc, Sdst, Sden, Scnt, bias)


def _selectors():
    ssrc = np.zeros((DIM, DIM), np.float32)
    sdst = np.zeros((DIM, DIM), np.float32)
    for hh in range(H):
        ssrc[hh, hh * DH:(hh + 1) * DH] = 1.0
        sdst[4 + hh, hh * DH:(hh + 1) * DH] = 1.0
    sden = np.zeros((32, DIM), np.float32)
    sden[0, 0:32] = 1.0
    sden[1, 32:64] = 1.0
    sden[16, 64:96] = 1.0
    sden[17, 96:128] = 1.0
    scnt = np.zeros((32, DIM), np.float32)
    scnt[2, :] = 1.0
    return ssrc, sdst, sden, scnt


_SSRC, _SDST, _SDEN, _SCNT = _selectors()


def kernel(x, edge_index, W_gat, att_src, att_dst, b_gat, W_sage_l, b_sage_l,
           W_sage_r, W_proj, b_proj, gamma, beta):
    E = edge_index.shape[1]
    E_pad = -(-E // (NTILES * 8 * EB)) * (NTILES * 8 * EB)

    xp = jnp.zeros((NP, DIM), jnp.float32).at[:N].set(x)

    A8 = jnp.zeros((DIM, DIM), jnp.float32)
    for hh in range(H):
        A8 = A8.at[hh * DH:(hh + 1) * DH, hh].set(att_src[hh])
        A8 = A8.at[hh * DH:(hh + 1) * DH, 4 + hh].set(att_dst[hh])

    h, a8, xr = _pre_call(xp, W_gat, A8, W_sage_r)

    pad = jnp.full((E_pad - E,), N, jnp.int32)
    esrc = jnp.concatenate([edge_index[0].astype(jnp.int32), pad])
    edst = jnp.concatenate([edge_index[1].astype(jnp.int32), pad])
    th = jnp.concatenate([h[:, :HALF], h[:, HALF:]], axis=0)
    tx = jnp.concatenate([xp[:, :HALF], xp[:, HALF:]], axis=0)
    atab = jnp.stack([
        jnp.stack([a8[:, 0], a8[:, 1], a8[:, 4], a8[:, 5]], axis=1).reshape(-1),
        jnp.stack([a8[:, 2], a8[:, 3], a8[:, 6], a8[:, 7]], axis=1).reshape(-1),
    ], axis=0)
    z64 = jnp.zeros((EB, HALF), jnp.float32)
    z16 = jnp.zeros((EB, 16), jnp.float32)

    msg2, dcnt2, nsum2 = _sc_call(esrc, edst, th, tx, atab, z64, z16)

    msg = jnp.concatenate([msg2[0], msg2[1]], axis=1)
    nsum = jnp.concatenate([nsum2[0], nsum2[1]], axis=1)
    dc = jnp.concatenate([dcnt2[0], dcnt2[1]], axis=1)

    bias = jnp.zeros((8, DIM), jnp.float32)
    bias = bias.at[0].set(b_gat).at[1].set(b_sage_l).at[2].set(b_proj)
    bias = bias.at[3].set(gamma).at[4].set(beta)

    out = _post_call(xp, h, a8, xr, msg, nsum, dc, W_sage_l,
                     W_proj[:DIM], W_proj[DIM:], _SSRC, _SDST, _SDEN, _SCNT,
                     bias)
    return out[:N]


# speedup vs baseline: 47.6873x; 1.1466x over previous
import functools

import jax
import jax.numpy as jnp
import numpy as np
from jax import lax
from jax.experimental import pallas as pl
from jax.experimental.pallas import tpu as pltpu
from jax.experimental.pallas import tpu_sc as plsc

N = 10000
DIM = 128
H = 4
DH = 32
HALF = 64
NP = 10048
NTILES = 16
NCORES = 2
EB = 128
ROWS_PER_TILE = NP // NTILES
RPT_FULL = ROWS_PER_TILE // 128
RPT_REM = ROWS_PER_TILE % 128


def _pre_body(x_ref, wg_ref, a8_ref, wsr_ref, h_ref, a8o_ref, xr_ref):
    xb = x_ref[...]
    hb = jnp.dot(xb, wg_ref[...], preferred_element_type=jnp.float32)
    h_ref[...] = hb
    a8o_ref[...] = jnp.dot(hb, a8_ref[...], preferred_element_type=jnp.float32)
    xr_ref[...] = jnp.dot(xb, wsr_ref[...], preferred_element_type=jnp.float32)


def _pre_call(xp, Wg, A8, Wsr):
    BR = 512
    full = pl.BlockSpec((DIM, DIM), lambda i: (0, 0))
    row = pl.BlockSpec((BR, DIM), lambda i: (i, 0))
    return pl.pallas_call(
        _pre_body,
        grid=(NP // BR,),
        in_specs=[row, full, full, full],
        out_specs=[row, row, row],
        out_shape=[jax.ShapeDtypeStruct((NP, DIM), jnp.float32)] * 3,
    )(xp, Wg, A8, Wsr)


def _sc_mesh():
    return plsc.VectorSubcoreMesh(
        core_axis_name="c", subcore_axis_name="s",
        num_cores=NCORES, num_subcores=NTILES)


_SC_PARAMS = pltpu.CompilerParams(
    needs_layout_passes=False, use_tc_tiling_on_sc=False)


def _sc_call(esrc, edst, th, tx, atab, z64, z16):
    E_pad = esrc.shape[0]
    EPT = E_pad // NTILES
    NM = EPT // EB

    @functools.partial(
        pl.kernel,
        out_type=[
            jax.ShapeDtypeStruct((NCORES, NP, HALF), jnp.float32),
            jax.ShapeDtypeStruct((NCORES, NP, 16), jnp.float32),
            jax.ShapeDtypeStruct((NCORES, NP, HALF), jnp.float32),
        ],
        mesh=_sc_mesh(),
        compiler_params=_SC_PARAMS,
        scratch_types=[
            pltpu.VMEM((NP * 4,), jnp.float32),
            pltpu.VMEM((8, EB), jnp.int32),
            pltpu.VMEM((8, EB), jnp.int32),
            pltpu.VMEM((8, EB), jnp.int32),
            pltpu.VMEM((4, EB, HALF), jnp.float32),
            pltpu.VMEM((2, EB, 16), jnp.float32),
            pltpu.VMEM_SHARED((NP, HALF), jnp.float32),
            pltpu.VMEM_SHARED((NP, 16), jnp.float32),
        ] + [pltpu.SemaphoreType.DMA] * 16,
    )
    def sc_kernel(esrc_r, edst_r, th_r, tx_r, atab_r, z64_r, z16_r,
                  msg_o, dcnt_o, nsum_o,
                  atab_v, srcv, gsrcv, dstv, xbuf, dcntb,
                  ACC, DCNT, *sems):
        isem = sems[0:8]
        gsem = sems[8:12]
        ssem = sems[12:16]
        c = lax.axis_index("c")
        s = lax.axis_index("s")
        coff = c * NP
        pltpu.sync_copy(atab_r.at[c], atab_v)
        base_rows = s * ROWS_PER_TILE
        for i in range(RPT_FULL):
            pltpu.sync_copy(z64_r, ACC.at[pl.ds(base_rows + i * EB, EB)])
            pltpu.sync_copy(z16_r, DCNT.at[pl.ds(base_rows + i * EB, EB)])
        pltpu.sync_copy(z64_r.at[pl.ds(0, RPT_REM)],
                        ACC.at[pl.ds(base_rows + RPT_FULL * EB, RPT_REM)])
        pltpu.sync_copy(z16_r.at[pl.ds(0, RPT_REM)],
                        DCNT.at[pl.ds(base_rows + RPT_FULL * EB, RPT_REM)])
        plsc.subcore_barrier()

        gdn = lax.GatherDimensionNumbers(
            offset_dims=(), collapsed_slice_dims=(0,), start_index_map=(0,))

        def bcast_lane(vec, j):
            jidx = jnp.full((16, 1), j, jnp.int32)
            return lax.gather(
                vec, jidx, gdn, (1,),
                mode=lax.GatherScatterMode.PROMISE_IN_BOUNDS)

        iota16 = lax.iota(jnp.int32, 16)
        col0 = jnp.zeros((16,), jnp.int32)
        col1 = jnp.full((16,), 1, jnp.int32)
        col2 = jnp.full((16,), 2, jnp.int32)
        ones_f = jnp.full((16,), 1.0, jnp.float32)
        tile_base = s * EPT

        def fire_idx(m, r):
            base = tile_base + m * EB
            pltpu.async_copy(esrc_r.at[pl.ds(base, EB)], srcv.at[r], isem[r])
            pltpu.async_copy(edst_r.at[pl.ds(base, EB)], dstv.at[r], isem[r])

        def drain_idx(m, r):
            base = tile_base + m * EB
            pltpu.make_async_copy(
                esrc_r.at[pl.ds(base, EB)], srcv.at[r], isem[r]).wait()
            pltpu.make_async_copy(
                edst_r.at[pl.ds(base, EB)], dstv.at[r], isem[r]).wait()

        def make_gsrc(r):
            for q in range(EB // 16):
                sl = pl.ds(q * 16, 16)
                gsrcv[r, sl] = srcv[r, sl] + coff

        def gat_compute(b, g):
            for k in range(EB // 16):
                sl = pl.ds(k * 16, 16)
                sv = srcv[b, sl]
                dv = dstv[b, sl]
                si = sv * 4
                di = dv * 4
                as0 = plsc.load_gather(atab_v, [si])
                as1 = plsc.load_gather(atab_v, [si + 1])
                ad0 = plsc.load_gather(atab_v, [di + 2])
                ad1 = plsc.load_gather(atab_v, [di + 3])
                z0 = as0 + ad0
                z1 = as1 + ad1
                w0 = jnp.exp(jnp.maximum(z0, 0.2 * z0))
                w1 = jnp.exp(jnp.maximum(z1, 0.2 * z1))
                rows = iota16 + (k * 16)
                plsc.store_scatter(dcntb.at[g], [rows, col0], w0)
                plsc.store_scatter(dcntb.at[g], [rows, col1], w1)
                plsc.store_scatter(dcntb.at[g], [rows, col2], ones_f)
                for j in range(16):
                    e = k * 16 + j
                    b0 = bcast_lane(w0, j)
                    b1 = bcast_lane(w1, j)
                    wm = xbuf.at[2 + g]
                    wm[e, pl.ds(0, 16)] = xbuf[g, e, pl.ds(0, 16)] * b0
                    wm[e, pl.ds(16, 16)] = xbuf[g, e, pl.ds(16, 16)] * b0
                    wm[e, pl.ds(32, 16)] = xbuf[g, e, pl.ds(32, 16)] * b1
                    wm[e, pl.ds(48, 16)] = xbuf[g, e, pl.ds(48, 16)] * b1

        def gat_drain_scat(g, row):
            pltpu.make_async_copy(
                xbuf.at[2 + g], ACC.at[dstv.at[row]], ssem[g]).wait()
            pltpu.make_async_copy(
                dcntb.at[g], DCNT.at[dstv.at[row]], ssem[g]).wait()

        NI = NM // 4

        fire_idx(0, 0)
        fire_idx(1, 1)
        drain_idx(0, 0)
        make_gsrc(0)
        pltpu.async_copy(th_r.at[gsrcv.at[0]], xbuf.at[0], gsem[0])

        def gat_body(i, carry):
            for b in range(4):
                m = 4 * i + b
                g = b % 2
                def a_block(bb=b):
                    drain_idx(m + 1, (bb + 1) % 4)
                    make_gsrc((bb + 1) % 4)
                    pltpu.async_copy(th_r.at[gsrcv.at[(bb + 1) % 4]],
                                     xbuf.at[(bb + 1) % 2], gsem[(bb + 1) % 2])
                if b < 3:
                    a_block()
                else:
                    @pl.when(i < NI - 1)
                    def _():
                        a_block()
                if b < 2:
                    fire_idx(m + 2, (b + 2) % 4)
                else:
                    @pl.when(i < NI - 1)
                    def _():
                        fire_idx(m + 2, (b + 2) % 4)
                pltpu.make_async_copy(
                    th_r.at[gsrcv.at[b]], xbuf.at[g], gsem[g]).wait()
                gat_compute(b, g)
                pltpu.async_copy(xbuf.at[2 + g], ACC.at[dstv.at[b]],
                                 ssem[g], add=True)
                pltpu.async_copy(dcntb.at[g], DCNT.at[dstv.at[b]],
                                 ssem[g], add=True)
                gat_drain_scat(g, b)
            return carry

        lax.fori_loop(0, NI, gat_body, 0)

        plsc.subcore_barrier()
        pltpu.sync_copy(ACC.at[pl.ds(base_rows, ROWS_PER_TILE)],
                        msg_o.at[c, pl.ds(base_rows, ROWS_PER_TILE)])
        pltpu.sync_copy(DCNT.at[pl.ds(base_rows, ROWS_PER_TILE)],
                        dcnt_o.at[c, pl.ds(base_rows, ROWS_PER_TILE)])
        for i in range(RPT_FULL):
            pltpu.sync_copy(z64_r, ACC.at[pl.ds(base_rows + i * EB, EB)])
        pltpu.sync_copy(z64_r.at[pl.ds(0, RPT_REM)],
                        ACC.at[pl.ds(base_rows + RPT_FULL * EB, RPT_REM)])
        plsc.subcore_barrier()

        def sage_drain_scat(q, row):
            pltpu.make_async_copy(
                xbuf.at[q], ACC.at[dstv.at[row]], ssem[q]).wait()

        NI2 = NM // 8

        fire_idx(0, 0)
        fire_idx(1, 1)
        drain_idx(0, 0)
        make_gsrc(0)
        pltpu.async_copy(tx_r.at[gsrcv.at[0]], xbuf.at[0], gsem[0])

        def sage_body(i, carry):
            for b in range(8):
                m = 8 * i + b
                q = b % 4
                def a_gather(bb=b):
                    drain_idx(m + 1, (bb + 1) % 8)
                    make_gsrc((bb + 1) % 8)
                    pltpu.async_copy(tx_r.at[gsrcv.at[(bb + 1) % 8]],
                                     xbuf.at[(bb + 1) % 4], gsem[(bb + 1) % 4])
                if b < 7:
                    a_gather()
                else:
                    @pl.when(i < NI2 - 1)
                    def _():
                        a_gather()
                if b < 6:
                    fire_idx(m + 2, (b + 2) % 8)
                else:
                    @pl.when(i < NI2 - 1)
                    def _():
                        fire_idx(m + 2, (b + 2) % 8)
                pltpu.make_async_copy(
                    tx_r.at[gsrcv.at[b]], xbuf.at[q], gsem[q]).wait()
                pltpu.async_copy(xbuf.at[q], ACC.at[dstv.at[b]],
                                 ssem[q], add=True)
                sage_drain_scat(q, b)
            return carry

        lax.fori_loop(0, NI2, sage_body, 0)

        plsc.subcore_barrier()
        pltpu.sync_copy(ACC.at[pl.ds(base_rows, ROWS_PER_TILE)],
                        nsum_o.at[c, pl.ds(base_rows, ROWS_PER_TILE)])

    return sc_kernel(esrc, edst, th, tx, atab, z64, z16)


def _post_body(xp_ref, h_ref, a8_ref, xr_ref, msg_ref, nsum_ref, dc_ref,
               wsl_ref, wpt_ref, wpb_ref, ssrc_ref, sdst_ref, sden_ref,
               scnt_ref, bias_ref, out_ref):
    a8b = a8_ref[...]
    asx = jnp.dot(a8b, ssrc_ref[...], preferred_element_type=jnp.float32)
    adx = jnp.dot(a8b, sdst_ref[...], preferred_element_type=jnp.float32)
    z = asx + adx
    wl = jnp.exp(jnp.maximum(z, 0.2 * z))
    hb = h_ref[...]
    msg_t = msg_ref[...] + hb * wl
    dcb = dc_ref[...]
    den = jnp.dot(dcb, sden_ref[...], preferred_element_type=jnp.float32) + wl + 1e-16
    gat = msg_t / den + bias_ref[0:1, :]
    cnt = jnp.maximum(jnp.dot(dcb, scnt_ref[...], preferred_element_type=jnp.float32), 1.0)
    mean = nsum_ref[...] / cnt
    sage = jnp.dot(mean, wsl_ref[...], preferred_element_type=jnp.float32) \
        + bias_ref[1:2, :] + xr_ref[...]
    o = jnp.dot(gat, wpt_ref[...], preferred_element_type=jnp.float32) \
        + jnp.dot(sage, wpb_ref[...], preferred_element_type=jnp.float32) \
        + bias_ref[2:3, :] + xp_ref[...]
    mu = jnp.mean(o, axis=-1, keepdims=True)
    d_ = o - mu
    var = jnp.mean(d_ * d_, axis=-1, keepdims=True)
    out_ref[...] = bias_ref[3:4, :] * (d_ * lax.rsqrt(var + 1e-5)) + bias_ref[4:5, :]


def _post_call(xp, h, a8, xr, msg, nsum, dc, Wsl, Wpt, Wpb, Ssrc, Sdst, Sden,
               Scnt, bias):
    BR = 512
    row = pl.BlockSpec((BR, DIM), lambda i: (i, 0))
    row32 = pl.BlockSpec((BR, 32), lambda i: (i, 0))
    full = pl.BlockSpec((DIM, DIM), lambda i: (0, 0))
    full32 = pl.BlockSpec((32, DIM), lambda i: (0, 0))
    fullb = pl.BlockSpec((8, DIM), lambda i: (0, 0))
    return pl.pallas_call(
        _post_body,
        grid=(NP // BR,),
        in_specs=[row, row, row, row, row, row, row32,
                  full, full, full, full, full, full32, full32, fullb],
        out_specs=row,
        out_shape=jax.ShapeDtypeStruct((NP, DIM), jnp.float32),
    )(xp, h, a8, xr, msg, nsum, dc, Wsl, Wpt, Wpb, Ssrc, Sdst, Sden, Scnt, bias)


def _selectors():
    ssrc = np.zeros((DIM, DIM), np.float32)
    sdst = np.zeros((DIM, DIM), np.float32)
    for hh in range(H):
        ssrc[hh, hh * DH:(hh + 1) * DH] = 1.0
        sdst[4 + hh, hh * DH:(hh + 1) * DH] = 1.0
    sden = np.zeros((32, DIM), np.float32)
    sden[0, 0:32] = 1.0
    sden[1, 32:64] = 1.0
    sden[16, 64:96] = 1.0
    sden[17, 96:128] = 1.0
    scnt = np.zeros((32, DIM), np.float32)
    scnt[2, :] = 1.0
    return ssrc, sdst, sden, scnt


_SSRC, _SDST, _SDEN, _SCNT = _selectors()


def kernel(x, edge_index, W_gat, att_src, att_dst, b_gat, W_sage_l, b_sage_l,
           W_sage_r, W_proj, b_proj, gamma, beta):
    E = edge_index.shape[1]
    E_pad = -(-E // (NTILES * 8 * EB)) * (NTILES * 8 * EB)

    xp = jnp.zeros((NP, DIM), jnp.float32).at[:N].set(x)

    A8 = jnp.zeros((DIM, DIM), jnp.float32)
    for hh in range(H):
        A8 = A8.at[hh * DH:(hh + 1) * DH, hh].set(att_src[hh])
        A8 = A8.at[hh * DH:(hh + 1) * DH, 4 + hh].set(att_dst[hh])

    h, a8, xr = _pre_call(xp, W_gat, A8, W_sage_r)

    pad = jnp.full((E_pad - E,), N, jnp.int32)
    esrc = jnp.concatenate([edge_index[0].astype(jnp.int32), pad])
    edst = jnp.concatenate([edge_index[1].astype(jnp.int32), pad])
    th = jnp.concatenate([h[:, :HALF], h[:, HALF:]], axis=0)
    tx = jnp.concatenate([xp[:, :HALF], xp[:, HALF:]], axis=0)
    atab = jnp.stack([
        jnp.stack([a8[:, 0], a8[:, 1], a8[:, 4], a8[:, 5]], axis=1).reshape(-1),
        jnp.stack([a8[:, 2], a8[:, 3], a8[:, 6], a8[:, 7]], axis=1).reshape(-1),
    ], axis=0)
    z64 = jnp.zeros((EB, HALF), jnp.float32)
    z16 = jnp.zeros((EB, 16), jnp.float32)

    msg2, dcnt2, nsum2 = _sc_call(esrc, edst, th, tx, atab, z64, z16)

    msg = jnp.concatenate([msg2[0], msg2[1]], axis=1)
    nsum = jnp.concatenate([nsum2[0], nsum2[1]], axis=1)
    dc = jnp.concatenate([dcnt2[0], dcnt2[1]], axis=1)

    bias = jnp.zeros((8, DIM), jnp.float32)
    bias = bias.at[0].set(b_gat).at[1].set(b_sage_l).at[2].set(b_proj)
    bias = bias.at[3].set(gamma).at[4].set(beta)

    out = _post_call(xp, h, a8, xr, msg, nsum, dc, W_sage_l,
                     W_proj[:DIM], W_proj[DIM:], _SSRC, _SDST, _SDEN, _SCNT,
                     bias)
    return out[:N]


# speedup vs baseline: 49.5243x; 1.0385x over previous
import functools

import jax
import jax.numpy as jnp
import numpy as np
from jax import lax
from jax.experimental import pallas as pl
from jax.experimental.pallas import tpu as pltpu
from jax.experimental.pallas import tpu_sc as plsc

N = 10000
DIM = 128
H = 4
DH = 32
HALF = 64
NP = 10048
NTILES = 16
NCORES = 2
EB = 128
ROWS_PER_TILE = NP // NTILES
RPT_FULL = ROWS_PER_TILE // 128
RPT_REM = ROWS_PER_TILE % 128


BR = 400


def _pre_body(x_ref, wg_ref, a8w_ref, wsr_ref, sel_ref,
              h_ref, a8o_ref, xr_ref, th_ref, tx_ref, atab_ref):
    c = pl.program_id(1)
    xb = x_ref[...]
    hb = jnp.dot(xb, wg_ref[...], preferred_element_type=jnp.float32)
    a8 = jnp.dot(hb, a8w_ref[...], preferred_element_type=jnp.float32)
    h_ref[...] = hb
    a8o_ref[...] = a8
    xr_ref[...] = jnp.dot(xb, wsr_ref[...], preferred_element_type=jnp.float32)
    th_ref[...] = jnp.where(c == 0, hb[:, :HALF], hb[:, HALF:])[None]
    tx_ref[...] = jnp.where(c == 0, xb[:, :HALF], xb[:, HALF:])[None]
    atab_ref[...] = jnp.dot(a8, sel_ref[0],
                            preferred_element_type=jnp.float32)[None]


def _pre_call(x, Wg, A8, Wsr, Sel):
    row = pl.BlockSpec((BR, DIM), lambda i, c: (i, 0))
    full = pl.BlockSpec((DIM, DIM), lambda i, c: (0, 0))
    halfo = pl.BlockSpec((1, BR, HALF), lambda i, c: (c, i, 0))
    return pl.pallas_call(
        _pre_body,
        grid=(N // BR, NCORES),
        in_specs=[row, full, full, full,
                  pl.BlockSpec((1, DIM, 4), lambda i, c: (c, 0, 0))],
        out_specs=[row, row, row, halfo, halfo,
                   pl.BlockSpec((1, BR, 4), lambda i, c: (c, i, 0))],
        out_shape=[
            jax.ShapeDtypeStruct((N, DIM), jnp.float32),
            jax.ShapeDtypeStruct((N, DIM), jnp.float32),
            jax.ShapeDtypeStruct((N, DIM), jnp.float32),
            jax.ShapeDtypeStruct((NCORES, NP, HALF), jnp.float32),
            jax.ShapeDtypeStruct((NCORES, NP, HALF), jnp.float32),
            jax.ShapeDtypeStruct((NCORES, NP, 4), jnp.float32),
        ],
    )(x, Wg, A8, Wsr, Sel)


def _sc_mesh():
    return plsc.VectorSubcoreMesh(
        core_axis_name="c", subcore_axis_name="s",
        num_cores=NCORES, num_subcores=NTILES)


_SC_PARAMS = pltpu.CompilerParams(
    needs_layout_passes=False, use_tc_tiling_on_sc=False)


def _sc_call(esrc, edst, th, tx, atab, z64, z16):
    E_pad = esrc.shape[0]
    EPT = E_pad // NTILES
    NM = EPT // EB

    @functools.partial(
        pl.kernel,
        out_type=[
            jax.ShapeDtypeStruct((NCORES, NP, HALF), jnp.float32),
            jax.ShapeDtypeStruct((NCORES, NP, 16), jnp.float32),
            jax.ShapeDtypeStruct((NCORES, NP, HALF), jnp.float32),
        ],
        mesh=_sc_mesh(),
        compiler_params=_SC_PARAMS,
        scratch_types=[
            pltpu.VMEM((NP * 4,), jnp.float32),
            pltpu.VMEM((8, EB), jnp.int32),
            pltpu.VMEM((8, EB), jnp.int32),
            pltpu.VMEM((8, EB), jnp.int32),
            pltpu.VMEM((4, EB, HALF), jnp.float32),
            pltpu.VMEM((2, EB, 16), jnp.float32),
            pltpu.VMEM_SHARED((NP, HALF), jnp.float32),
            pltpu.VMEM_SHARED((NP, 16), jnp.float32),
        ] + [pltpu.SemaphoreType.DMA] * 16,
    )
    def sc_kernel(esrc_r, edst_r, th_r, tx_r, atab_r, z64_r, z16_r,
                  msg_o, dcnt_o, nsum_o,
                  atab_v, srcv, gsrcv, dstv, xbuf, dcntb,
                  ACC, DCNT, *sems):
        isem = sems[0:8]
        gsem = sems[8:12]
        ssem = sems[12:16]
        c = lax.axis_index("c")
        s = lax.axis_index("s")
        coff = c * NP
        pltpu.sync_copy(atab_r.at[c], atab_v)
        base_rows = s * ROWS_PER_TILE
        for i in range(RPT_FULL):
            pltpu.sync_copy(z64_r, ACC.at[pl.ds(base_rows + i * EB, EB)])
            pltpu.sync_copy(z16_r, DCNT.at[pl.ds(base_rows + i * EB, EB)])
        pltpu.sync_copy(z64_r.at[pl.ds(0, RPT_REM)],
                        ACC.at[pl.ds(base_rows + RPT_FULL * EB, RPT_REM)])
        pltpu.sync_copy(z16_r.at[pl.ds(0, RPT_REM)],
                        DCNT.at[pl.ds(base_rows + RPT_FULL * EB, RPT_REM)])
        plsc.subcore_barrier()

        gdn = lax.GatherDimensionNumbers(
            offset_dims=(), collapsed_slice_dims=(0,), start_index_map=(0,))

        def bcast_lane(vec, j):
            jidx = jnp.full((16, 1), j, jnp.int32)
            return lax.gather(
                vec, jidx, gdn, (1,),
                mode=lax.GatherScatterMode.PROMISE_IN_BOUNDS)

        iota16 = lax.iota(jnp.int32, 16)
        col0 = jnp.zeros((16,), jnp.int32)
        col1 = jnp.full((16,), 1, jnp.int32)
        col2 = jnp.full((16,), 2, jnp.int32)
        ones_f = jnp.full((16,), 1.0, jnp.float32)
        tile_base = s * EPT

        def fire_idx(m, r):
            base = tile_base + m * EB
            pltpu.async_copy(esrc_r.at[pl.ds(base, EB)], srcv.at[r], isem[r])
            pltpu.async_copy(edst_r.at[pl.ds(base, EB)], dstv.at[r], isem[r])

        def drain_idx(m, r):
            base = tile_base + m * EB
            pltpu.make_async_copy(
                esrc_r.at[pl.ds(base, EB)], srcv.at[r], isem[r]).wait()
            pltpu.make_async_copy(
                edst_r.at[pl.ds(base, EB)], dstv.at[r], isem[r]).wait()

        def make_gsrc(r):
            for q in range(EB // 16):
                sl = pl.ds(q * 16, 16)
                gsrcv[r, sl] = srcv[r, sl] + coff

        def gat_compute(b, g):
            for k in range(EB // 16):
                sl = pl.ds(k * 16, 16)
                sv = srcv[b, sl]
                dv = dstv[b, sl]
                si = sv * 4
                di = dv * 4
                as0 = plsc.load_gather(atab_v, [si])
                as1 = plsc.load_gather(atab_v, [si + 1])
                ad0 = plsc.load_gather(atab_v, [di + 2])
                ad1 = plsc.load_gather(atab_v, [di + 3])
                z0 = as0 + ad0
                z1 = as1 + ad1
                w0 = jnp.exp(jnp.maximum(z0, 0.2 * z0))
                w1 = jnp.exp(jnp.maximum(z1, 0.2 * z1))
                rows = iota16 + (k * 16)
                plsc.store_scatter(dcntb.at[g], [rows, col0], w0)
                plsc.store_scatter(dcntb.at[g], [rows, col1], w1)
                plsc.store_scatter(dcntb.at[g], [rows, col2], ones_f)
                for j in range(16):
                    e = k * 16 + j
                    b0 = bcast_lane(w0, j)
                    b1 = bcast_lane(w1, j)
                    wm = xbuf.at[2 + g]
                    wm[e, pl.ds(0, 16)] = xbuf[g, e, pl.ds(0, 16)] * b0
                    wm[e, pl.ds(16, 16)] = xbuf[g, e, pl.ds(16, 16)] * b0
                    wm[e, pl.ds(32, 16)] = xbuf[g, e, pl.ds(32, 16)] * b1
                    wm[e, pl.ds(48, 16)] = xbuf[g, e, pl.ds(48, 16)] * b1

        def gat_drain_scat(g, row):
            pltpu.make_async_copy(
                xbuf.at[2 + g], ACC.at[dstv.at[row]], ssem[g]).wait()
            pltpu.make_async_copy(
                dcntb.at[g], DCNT.at[dstv.at[row]], ssem[g]).wait()

        NI = NM // 4

        fire_idx(0, 0)
        fire_idx(1, 1)
        drain_idx(0, 0)
        make_gsrc(0)
        pltpu.async_copy(th_r.at[gsrcv.at[0]], xbuf.at[0], gsem[0])

        def gat_body(i, carry):
            for b in range(4):
                m = 4 * i + b
                g = b % 2
                def a_block(bb=b):
                    drain_idx(m + 1, (bb + 1) % 4)
                    make_gsrc((bb + 1) % 4)
                    pltpu.async_copy(th_r.at[gsrcv.at[(bb + 1) % 4]],
                                     xbuf.at[(bb + 1) % 2], gsem[(bb + 1) % 2])
                if b < 3:
                    a_block()
                else:
                    @pl.when(i < NI - 1)
                    def _():
                        a_block()
                if b < 2:
                    fire_idx(m + 2, (b + 2) % 4)
                else:
                    @pl.when(i < NI - 1)
                    def _():
                        fire_idx(m + 2, (b + 2) % 4)
                pltpu.make_async_copy(
                    th_r.at[gsrcv.at[b]], xbuf.at[g], gsem[g]).wait()
                gat_compute(b, g)
                pltpu.async_copy(xbuf.at[2 + g], ACC.at[dstv.at[b]],
                                 ssem[g], add=True)
                pltpu.async_copy(dcntb.at[g], DCNT.at[dstv.at[b]],
                                 ssem[g], add=True)
                gat_drain_scat(g, b)
            return carry

        lax.fori_loop(0, NI, gat_body, 0)

        plsc.subcore_barrier()
        pltpu.sync_copy(ACC.at[pl.ds(base_rows, ROWS_PER_TILE)],
                        msg_o.at[c, pl.ds(base_rows, ROWS_PER_TILE)])
        pltpu.sync_copy(DCNT.at[pl.ds(base_rows, ROWS_PER_TILE)],
                        dcnt_o.at[c, pl.ds(base_rows, ROWS_PER_TILE)])
        for i in range(RPT_FULL):
            pltpu.sync_copy(z64_r, ACC.at[pl.ds(base_rows + i * EB, EB)])
        pltpu.sync_copy(z64_r.at[pl.ds(0, RPT_REM)],
                        ACC.at[pl.ds(base_rows + RPT_FULL * EB, RPT_REM)])
        plsc.subcore_barrier()

        def sage_drain_scat(q, row):
            pltpu.make_async_copy(
                xbuf.at[q], ACC.at[dstv.at[row]], ssem[q]).wait()

        NI2 = NM // 8

        fire_idx(0, 0)
        fire_idx(1, 1)
        drain_idx(0, 0)
        make_gsrc(0)
        pltpu.async_copy(tx_r.at[gsrcv.at[0]], xbuf.at[0], gsem[0])

        def sage_body(i, carry):
            for b in range(8):
                m = 8 * i + b
                q = b % 4
                def a_gather(bb=b):
                    drain_idx(m + 1, (bb + 1) % 8)
                    make_gsrc((bb + 1) % 8)
                    pltpu.async_copy(tx_r.at[gsrcv.at[(bb + 1) % 8]],
                                     xbuf.at[(bb + 1) % 4], gsem[(bb + 1) % 4])
                if b < 7:
                    a_gather()
                else:
                    @pl.when(i < NI2 - 1)
                    def _():
                        a_gather()
                if b < 6:
                    fire_idx(m + 2, (b + 2) % 8)
                else:
                    @pl.when(i < NI2 - 1)
                    def _():
                        fire_idx(m + 2, (b + 2) % 8)
                pltpu.make_async_copy(
                    tx_r.at[gsrcv.at[b]], xbuf.at[q], gsem[q]).wait()
                pltpu.async_copy(xbuf.at[q], ACC.at[dstv.at[b]],
                                 ssem[q], add=True)
                sage_drain_scat(q, b)
            return carry

        lax.fori_loop(0, NI2, sage_body, 0)

        plsc.subcore_barrier()
        pltpu.sync_copy(ACC.at[pl.ds(base_rows, ROWS_PER_TILE)],
                        nsum_o.at[c, pl.ds(base_rows, ROWS_PER_TILE)])

    return sc_kernel(esrc, edst, th, tx, atab, z64, z16)


def _post_body(x_ref, h_ref, a8_ref, xr_ref, m0_ref, m1_ref, n0_ref, n1_ref,
               d0_ref, d1_ref, wsl0_ref, wsl1_ref, wpt0_ref, wpt1_ref,
               wpb0_ref, wpb1_ref, ssrc_ref, sdst_ref, sden_ref, scnt_ref,
               bias_ref, out_ref):
    a8b = a8_ref[...]
    z = jnp.dot(a8b, ssrc_ref[...], preferred_element_type=jnp.float32) \
        + jnp.dot(a8b, sdst_ref[...], preferred_element_type=jnp.float32)
    wl = jnp.exp(jnp.maximum(z, 0.2 * z))
    hb = h_ref[...]
    d0b = d0_ref[0]
    d1b = d1_ref[0]
    gh = []
    for cc, (mref, db) in enumerate(((m0_ref, d0b), (m1_ref, d1b))):
        lo = cc * HALF
        wlh = wl[:, lo:lo + HALF]
        msg_t = mref[0] + hb[:, lo:lo + HALF] * wlh
        den = jnp.dot(db, sden_ref[...], preferred_element_type=jnp.float32) \
            + wlh + 1e-16
        gh.append(msg_t / den + bias_ref[0:1, lo:lo + HALF])
    cnt = jnp.maximum(
        jnp.dot(d0b, scnt_ref[...], preferred_element_type=jnp.float32), 1.0)
    sage = jnp.dot(n0_ref[0] / cnt[:, :HALF], wsl0_ref[...],
                   preferred_element_type=jnp.float32) \
        + jnp.dot(n1_ref[0] / cnt[:, HALF:], wsl1_ref[...],
                  preferred_element_type=jnp.float32) \
        + bias_ref[1:2, :] + xr_ref[...]
    o = jnp.dot(gh[0], wpt0_ref[...], preferred_element_type=jnp.float32) \
        + jnp.dot(gh[1], wpt1_ref[...], preferred_element_type=jnp.float32) \
        + jnp.dot(sage[:, :HALF], wpb0_ref[...], preferred_element_type=jnp.float32) \
        + jnp.dot(sage[:, HALF:], wpb1_ref[...], preferred_element_type=jnp.float32) \
        + bias_ref[2:3, :] + x_ref[...]
    mu = jnp.mean(o, axis=-1, keepdims=True)
    d_ = o - mu
    var = jnp.mean(d_ * d_, axis=-1, keepdims=True)
    out_ref[...] = bias_ref[3:4, :] * (d_ * lax.rsqrt(var + 1e-5)) + bias_ref[4:5, :]


def _post_call(x, h, a8, xr, msg2, nsum2, dcnt2, Wsl, Wpt, Wpb, Ssrc, Sdst,
               Sden, Scnt, bias):
    row = pl.BlockSpec((BR, DIM), lambda i: (i, 0))
    half0 = pl.BlockSpec((1, BR, HALF), lambda i: (0, i, 0))
    half1 = pl.BlockSpec((1, BR, HALF), lambda i: (1, i, 0))
    d16_0 = pl.BlockSpec((1, BR, 16), lambda i: (0, i, 0))
    d16_1 = pl.BlockSpec((1, BR, 16), lambda i: (1, i, 0))
    whalf = pl.BlockSpec((HALF, DIM), lambda i: (0, 0))
    full = pl.BlockSpec((DIM, DIM), lambda i: (0, 0))
    s16 = pl.BlockSpec((16, HALF), lambda i: (0, 0))
    s16c = pl.BlockSpec((16, DIM), lambda i: (0, 0))
    fullb = pl.BlockSpec((8, DIM), lambda i: (0, 0))
    return pl.pallas_call(
        _post_body,
        grid=(N // BR,),
        in_specs=[row, row, row, row, half0, half1, half0, half1,
                  d16_0, d16_1, whalf, whalf, whalf, whalf, whalf, whalf,
                  full, full, s16, s16c, fullb],
        out_specs=row,
        out_shape=jax.ShapeDtypeStruct((N, DIM), jnp.float32),
    )(x, h, a8, xr, msg2, msg2, nsum2, nsum2, dcnt2, dcnt2,
      Wsl[:HALF], Wsl[HALF:], Wpt[:HALF], Wpt[HALF:], Wpb[:HALF], Wpb[HALF:],
      Ssrc, Sdst, Sden, Scnt, bias)


def _selectors():
    ssrc = np.zeros((DIM, DIM), np.float32)
    sdst = np.zeros((DIM, DIM), np.float32)
    for hh in range(H):
        ssrc[hh, hh * DH:(hh + 1) * DH] = 1.0
        sdst[4 + hh, hh * DH:(hh + 1) * DH] = 1.0
    sden = np.zeros((16, HALF), np.float32)
    sden[0, 0:DH] = 1.0
    sden[1, DH:2 * DH] = 1.0
    scnt = np.zeros((16, DIM), np.float32)
    scnt[2, :] = 1.0
    sel = np.zeros((NCORES, DIM, 4), np.float32)
    for cdx in range(NCORES):
        sel[cdx, 2 * cdx, 0] = 1.0
        sel[cdx, 2 * cdx + 1, 1] = 1.0
        sel[cdx, 4 + 2 * cdx, 2] = 1.0
        sel[cdx, 5 + 2 * cdx, 3] = 1.0
    return ssrc, sdst, sden, scnt, sel


_SSRC, _SDST, _SDEN, _SCNT, _SEL = _selectors()


def kernel(x, edge_index, W_gat, att_src, att_dst, b_gat, W_sage_l, b_sage_l,
           W_sage_r, W_proj, b_proj, gamma, beta):
    E = edge_index.shape[1]
    E_pad = -(-E // (NTILES * 8 * EB)) * (NTILES * 8 * EB)

    A8 = jnp.zeros((DIM, DIM), jnp.float32)
    for hh in range(H):
        A8 = A8.at[hh * DH:(hh + 1) * DH, hh].set(att_src[hh])
        A8 = A8.at[hh * DH:(hh + 1) * DH, 4 + hh].set(att_dst[hh])

    h, a8, xr, th3, tx3, atab3 = _pre_call(x, W_gat, A8, W_sage_r,
                                           jnp.asarray(_SEL))

    pad = jnp.full((E_pad - E,), N, jnp.int32)
    esrc = jnp.concatenate([edge_index[0].astype(jnp.int32), pad])
    edst = jnp.concatenate([edge_index[1].astype(jnp.int32), pad])
    th = th3.reshape(NCORES * NP, HALF)
    tx = tx3.reshape(NCORES * NP, HALF)
    atab = atab3.reshape(NCORES, 4 * NP)
    z64 = jnp.zeros((EB, HALF), jnp.float32)
    z16 = jnp.zeros((EB, 16), jnp.float32)

    msg2, dcnt2, nsum2 = _sc_call(esrc, edst, th, tx, atab, z64, z16)

    bias = jnp.zeros((8, DIM), jnp.float32)
    bias = bias.at[0].set(b_gat).at[1].set(b_sage_l).at[2].set(b_proj)
    bias = bias.at[3].set(gamma).at[4].set(beta)

    return _post_call(x, h, a8, xr, msg2, nsum2, dcnt2, W_sage_l,
                      W_proj[:DIM], W_proj[DIM:], jnp.asarray(_SSRC),
                      jnp.asarray(_SDST), jnp.asarray(_SDEN),
                      jnp.asarray(_SCNT), bias)


# speedup vs baseline: 50.8780x; 1.0273x over previous
import functools

import jax
import jax.numpy as jnp
import numpy as np
from jax import lax
from jax.experimental import pallas as pl
from jax.experimental.pallas import tpu as pltpu
from jax.experimental.pallas import tpu_sc as plsc

N = 10000
DIM = 128
H = 4
DH = 32
HALF = 64
NP = 10048
NTILES = 16
NCORES = 2
EB = 128
ROWS_PER_TILE = NP // NTILES
RPT_FULL = ROWS_PER_TILE // 128
RPT_REM = ROWS_PER_TILE % 128


BR = 400


def _pre_body(x_ref, wg_ref, a8w_ref, wsr_ref, sel_ref,
              h_ref, a8o_ref, xr_ref, th_ref, tx_ref, atab_ref):
    c = pl.program_id(1)
    xb = x_ref[...]
    hb = jnp.dot(xb, wg_ref[...], preferred_element_type=jnp.float32)
    a8 = jnp.dot(hb, a8w_ref[...], preferred_element_type=jnp.float32)
    h_ref[...] = hb
    a8o_ref[...] = a8
    xr_ref[...] = jnp.dot(xb, wsr_ref[...], preferred_element_type=jnp.float32)
    th_ref[...] = jnp.where(c == 0, hb[:, :HALF], hb[:, HALF:])[None]
    tx_ref[...] = jnp.where(c == 0, xb[:, :HALF], xb[:, HALF:])[None]
    atab_ref[...] = jnp.dot(a8, sel_ref[0],
                            preferred_element_type=jnp.float32)[None]


def _pre_call(x, Wg, A8, Wsr, Sel):
    row = pl.BlockSpec((BR, DIM), lambda i, c: (i, 0))
    full = pl.BlockSpec((DIM, DIM), lambda i, c: (0, 0))
    halfo = pl.BlockSpec((1, BR, HALF), lambda i, c: (c, i, 0))
    return pl.pallas_call(
        _pre_body,
        grid=(N // BR, NCORES),
        in_specs=[row, full, full, full,
                  pl.BlockSpec((1, DIM, 4), lambda i, c: (c, 0, 0))],
        out_specs=[row, row, row, halfo, halfo,
                   pl.BlockSpec((1, BR, 4), lambda i, c: (c, i, 0))],
        out_shape=[
            jax.ShapeDtypeStruct((N, DIM), jnp.float32),
            jax.ShapeDtypeStruct((N, DIM), jnp.float32),
            jax.ShapeDtypeStruct((N, DIM), jnp.float32),
            jax.ShapeDtypeStruct((NCORES, NP, HALF), jnp.float32),
            jax.ShapeDtypeStruct((NCORES, NP, HALF), jnp.float32),
            jax.ShapeDtypeStruct((NCORES, NP, 4), jnp.float32),
        ],
    )(x, Wg, A8, Wsr, Sel)


def _sc_mesh():
    return plsc.VectorSubcoreMesh(
        core_axis_name="c", subcore_axis_name="s",
        num_cores=NCORES, num_subcores=NTILES)


_SC_PARAMS = pltpu.CompilerParams(
    needs_layout_passes=False, use_tc_tiling_on_sc=False)


def _sc_call(esrc, edst, th, tx, atab, z64, z16):
    E_pad = esrc.shape[0]
    EPT = E_pad // NTILES
    NM = EPT // EB

    @functools.partial(
        pl.kernel,
        out_type=[
            jax.ShapeDtypeStruct((NCORES, NP, HALF), jnp.float32),
            jax.ShapeDtypeStruct((NCORES, NP, 16), jnp.float32),
            jax.ShapeDtypeStruct((NCORES, NP, HALF), jnp.float32),
        ],
        mesh=_sc_mesh(),
        compiler_params=_SC_PARAMS,
        scratch_types=[
            pltpu.VMEM((NP * 4,), jnp.float32),
            pltpu.VMEM((8, EB), jnp.int32),
            pltpu.VMEM((8, EB), jnp.int32),
            pltpu.VMEM((8, EB), jnp.int32),
            pltpu.VMEM((4, EB, HALF), jnp.float32),
            pltpu.VMEM((2, EB, 16), jnp.float32),
            pltpu.VMEM_SHARED((NP, HALF), jnp.float32),
            pltpu.VMEM_SHARED((NP, 16), jnp.float32),
        ] + [pltpu.SemaphoreType.DMA] * 16,
    )
    def sc_kernel(esrc_r, edst_r, th_r, tx_r, atab_r, z64_r, z16_r,
                  msg_o, dcnt_o, nsum_o,
                  atab_v, srcv, gsrcv, dstv, xbuf, dcntb,
                  ACC, DCNT, *sems):
        isem = sems[0:8]
        gsem = sems[8:12]
        ssem = sems[12:16]
        c = lax.axis_index("c")
        s = lax.axis_index("s")
        coff = c * NP
        pltpu.sync_copy(atab_r.at[c], atab_v)
        base_rows = s * ROWS_PER_TILE
        for i in range(RPT_FULL):
            pltpu.sync_copy(z64_r, ACC.at[pl.ds(base_rows + i * EB, EB)])
            pltpu.sync_copy(z16_r, DCNT.at[pl.ds(base_rows + i * EB, EB)])
        pltpu.sync_copy(z64_r.at[pl.ds(0, RPT_REM)],
                        ACC.at[pl.ds(base_rows + RPT_FULL * EB, RPT_REM)])
        pltpu.sync_copy(z16_r.at[pl.ds(0, RPT_REM)],
                        DCNT.at[pl.ds(base_rows + RPT_FULL * EB, RPT_REM)])
        plsc.subcore_barrier()

        gdn = lax.GatherDimensionNumbers(
            offset_dims=(), collapsed_slice_dims=(0,), start_index_map=(0,))

        def bcast_lane(vec, j):
            jidx = jnp.full((16, 1), j, jnp.int32)
            return lax.gather(
                vec, jidx, gdn, (1,),
                mode=lax.GatherScatterMode.PROMISE_IN_BOUNDS)

        iota16 = lax.iota(jnp.int32, 16)
        col0 = jnp.zeros((16,), jnp.int32)
        col1 = jnp.full((16,), 1, jnp.int32)
        col2 = jnp.full((16,), 2, jnp.int32)
        ones_f = jnp.full((16,), 1.0, jnp.float32)
        tile_base = s * EPT

        def fire_idx(m, r):
            base = tile_base + m * EB
            pltpu.async_copy(esrc_r.at[pl.ds(base, EB)], srcv.at[r], isem[r])
            pltpu.async_copy(edst_r.at[pl.ds(base, EB)], dstv.at[r], isem[r])

        def drain_idx(m, r):
            base = tile_base + m * EB
            pltpu.make_async_copy(
                esrc_r.at[pl.ds(base, EB)], srcv.at[r], isem[r]).wait()
            pltpu.make_async_copy(
                edst_r.at[pl.ds(base, EB)], dstv.at[r], isem[r]).wait()

        def make_gsrc(r):
            for q in range(EB // 16):
                sl = pl.ds(q * 16, 16)
                gsrcv[r, sl] = srcv[r, sl] + coff

        def gat_compute(b, g):
            for k in range(EB // 16):
                sl = pl.ds(k * 16, 16)
                sv = srcv[b, sl]
                dv = dstv[b, sl]
                si = sv * 4
                di = dv * 4
                as0 = plsc.load_gather(atab_v, [si])
                as1 = plsc.load_gather(atab_v, [si + 1])
                ad0 = plsc.load_gather(atab_v, [di + 2])
                ad1 = plsc.load_gather(atab_v, [di + 3])
                z0 = as0 + ad0
                z1 = as1 + ad1
                w0 = jnp.exp(jnp.maximum(z0, 0.2 * z0))
                w1 = jnp.exp(jnp.maximum(z1, 0.2 * z1))
                rows = iota16 + (k * 16)
                plsc.store_scatter(dcntb.at[g], [rows, col0], w0)
                plsc.store_scatter(dcntb.at[g], [rows, col1], w1)
                plsc.store_scatter(dcntb.at[g], [rows, col2], ones_f)
                for j in range(16):
                    e = k * 16 + j
                    b0 = bcast_lane(w0, j)
                    b1 = bcast_lane(w1, j)
                    wm = xbuf.at[2 + g]
                    wm[e, pl.ds(0, 16)] = xbuf[g, e, pl.ds(0, 16)] * b0
                    wm[e, pl.ds(16, 16)] = xbuf[g, e, pl.ds(16, 16)] * b0
                    wm[e, pl.ds(32, 16)] = xbuf[g, e, pl.ds(32, 16)] * b1
                    wm[e, pl.ds(48, 16)] = xbuf[g, e, pl.ds(48, 16)] * b1

        def gat_drain_scat(g, row):
            pltpu.make_async_copy(
                xbuf.at[2 + g], ACC.at[dstv.at[row]], ssem[g]).wait()
            pltpu.make_async_copy(
                dcntb.at[g], DCNT.at[dstv.at[row]], ssem[g]).wait()

        NI = NM // 4

        fire_idx(0, 0)
        fire_idx(1, 1)
        drain_idx(0, 0)
        make_gsrc(0)
        pltpu.async_copy(th_r.at[gsrcv.at[0]], xbuf.at[0], gsem[0])

        def gat_body(i, carry):
            for b in range(4):
                m = 4 * i + b
                g = b % 2
                def a_block(bb=b):
                    drain_idx(m + 1, (bb + 1) % 4)
                    make_gsrc((bb + 1) % 4)
                    pltpu.async_copy(th_r.at[gsrcv.at[(bb + 1) % 4]],
                                     xbuf.at[(bb + 1) % 2], gsem[(bb + 1) % 2])
                if b < 3:
                    a_block()
                else:
                    @pl.when(i < NI - 1)
                    def _():
                        a_block()
                if b < 2:
                    fire_idx(m + 2, (b + 2) % 4)
                else:
                    @pl.when(i < NI - 1)
                    def _():
                        fire_idx(m + 2, (b + 2) % 4)
                if b > 0:
                    gat_drain_scat((g + 1) % 2, (b + 3) % 4)
                else:
                    @pl.when(i > 0)
                    def _():
                        gat_drain_scat((g + 1) % 2, (b + 3) % 4)
                pltpu.make_async_copy(
                    th_r.at[gsrcv.at[b]], xbuf.at[g], gsem[g]).wait()
                gat_compute(b, g)
                pltpu.async_copy(xbuf.at[2 + g], ACC.at[dstv.at[b]],
                                 ssem[g], add=True)
                pltpu.async_copy(dcntb.at[g], DCNT.at[dstv.at[b]],
                                 ssem[g], add=True)
            return carry

        lax.fori_loop(0, NI, gat_body, 0)
        gat_drain_scat(1, 3)

        plsc.subcore_barrier()
        pltpu.sync_copy(ACC.at[pl.ds(base_rows, ROWS_PER_TILE)],
                        msg_o.at[c, pl.ds(base_rows, ROWS_PER_TILE)])
        pltpu.sync_copy(DCNT.at[pl.ds(base_rows, ROWS_PER_TILE)],
                        dcnt_o.at[c, pl.ds(base_rows, ROWS_PER_TILE)])
        for i in range(RPT_FULL):
            pltpu.sync_copy(z64_r, ACC.at[pl.ds(base_rows + i * EB, EB)])
        pltpu.sync_copy(z64_r.at[pl.ds(0, RPT_REM)],
                        ACC.at[pl.ds(base_rows + RPT_FULL * EB, RPT_REM)])
        plsc.subcore_barrier()

        def sage_drain_scat(q, row):
            pltpu.make_async_copy(
                xbuf.at[q], ACC.at[dstv.at[row]], ssem[q]).wait()

        NI2 = NM // 8

        fire_idx(0, 0)
        fire_idx(1, 1)
        drain_idx(0, 0)
        make_gsrc(0)
        pltpu.async_copy(tx_r.at[gsrcv.at[0]], xbuf.at[0], gsem[0])

        def sage_body(i, carry):
            for b in range(8):
                m = 8 * i + b
                q = b % 4
                def a_gather(bb=b):
                    drain_idx(m + 1, (bb + 1) % 8)
                    make_gsrc((bb + 1) % 8)
                    pltpu.async_copy(tx_r.at[gsrcv.at[(bb + 1) % 8]],
                                     xbuf.at[(bb + 1) % 4], gsem[(bb + 1) % 4])
                if b < 7:
                    a_gather()
                else:
                    @pl.when(i < NI2 - 1)
                    def _():
                        a_gather()
                if b < 6:
                    fire_idx(m + 2, (b + 2) % 8)
                else:
                    @pl.when(i < NI2 - 1)
                    def _():
                        fire_idx(m + 2, (b + 2) % 8)
                if b > 0:
                    sage_drain_scat((q + 3) % 4, (b + 7) % 8)
                else:
                    @pl.when(i > 0)
                    def _():
                        sage_drain_scat((q + 3) % 4, (b + 7) % 8)
                pltpu.make_async_copy(
                    tx_r.at[gsrcv.at[b]], xbuf.at[q], gsem[q]).wait()
                pltpu.async_copy(xbuf.at[q], ACC.at[dstv.at[b]],
                                 ssem[q], add=True)
            return carry

        lax.fori_loop(0, NI2, sage_body, 0)
        sage_drain_scat(3, 7)

        plsc.subcore_barrier()
        pltpu.sync_copy(ACC.at[pl.ds(base_rows, ROWS_PER_TILE)],
                        nsum_o.at[c, pl.ds(base_rows, ROWS_PER_TILE)])

    return sc_kernel(esrc, edst, th, tx, atab, z64, z16)


def _post_body(x_ref, h_ref, a8_ref, xr_ref, m0_ref, m1_ref, n0_ref, n1_ref,
               d0_ref, d1_ref, wsl0_ref, wsl1_ref, wpt0_ref, wpt1_ref,
               wpb0_ref, wpb1_ref, ssrc_ref, sdst_ref, sden_ref, scnt_ref,
               bias_ref, out_ref):
    a8b = a8_ref[...]
    z = jnp.dot(a8b, ssrc_ref[...], preferred_element_type=jnp.float32) \
        + jnp.dot(a8b, sdst_ref[...], preferred_element_type=jnp.float32)
    wl = jnp.exp(jnp.maximum(z, 0.2 * z))
    hb = h_ref[...]
    d0b = d0_ref[0]
    d1b = d1_ref[0]
    gh = []
    for cc, (mref, db) in enumerate(((m0_ref, d0b), (m1_ref, d1b))):
        lo = cc * HALF
        wlh = wl[:, lo:lo + HALF]
        msg_t = mref[0] + hb[:, lo:lo + HALF] * wlh
        den = jnp.dot(db, sden_ref[...], preferred_element_type=jnp.float32) \
            + wlh + 1e-16
        gh.append(msg_t / den + bias_ref[0:1, lo:lo + HALF])
    cnt = jnp.maximum(
        jnp.dot(d0b, scnt_ref[...], preferred_element_type=jnp.float32), 1.0)
    sage = jnp.dot(n0_ref[0] / cnt[:, :HALF], wsl0_ref[...],
                   preferred_element_type=jnp.float32) \
        + jnp.dot(n1_ref[0] / cnt[:, HALF:], wsl1_ref[...],
                  preferred_element_type=jnp.float32) \
        + bias_ref[1:2, :] + xr_ref[...]
    o = jnp.dot(gh[0], wpt0_ref[...], preferred_element_type=jnp.float32) \
        + jnp.dot(gh[1], wpt1_ref[...], preferred_element_type=jnp.float32) \
        + jnp.dot(sage[:, :HALF], wpb0_ref[...], preferred_element_type=jnp.float32) \
        + jnp.dot(sage[:, HALF:], wpb1_ref[...], preferred_element_type=jnp.float32) \
        + bias_ref[2:3, :] + x_ref[...]
    mu = jnp.mean(o, axis=-1, keepdims=True)
    d_ = o - mu
    var = jnp.mean(d_ * d_, axis=-1, keepdims=True)
    out_ref[...] = bias_ref[3:4, :] * (d_ * lax.rsqrt(var + 1e-5)) + bias_ref[4:5, :]


def _post_call(x, h, a8, xr, msg2, nsum2, dcnt2, Wsl, Wpt, Wpb, Ssrc, Sdst,
               Sden, Scnt, bias):
    row = pl.BlockSpec((BR, DIM), lambda i: (i, 0))
    half0 = pl.BlockSpec((1, BR, HALF), lambda i: (0, i, 0))
    half1 = pl.BlockSpec((1, BR, HALF), lambda i: (1, i, 0))
    d16_0 = pl.BlockSpec((1, BR, 16), lambda i: (0, i, 0))
    d16_1 = pl.BlockSpec((1, BR, 16), lambda i: (1, i, 0))
    whalf = pl.BlockSpec((HALF, DIM), lambda i: (0, 0))
    full = pl.BlockSpec((DIM, DIM), lambda i: (0, 0))
    s16 = pl.BlockSpec((16, HALF), lambda i: (0, 0))
    s16c = pl.BlockSpec((16, DIM), lambda i: (0, 0))
    fullb = pl.BlockSpec((8, DIM), lambda i: (0, 0))
    return pl.pallas_call(
        _post_body,
        grid=(N // BR,),
        in_specs=[row, row, row, row, half0, half1, half0, half1,
                  d16_0, d16_1, whalf, whalf, whalf, whalf, whalf, whalf,
                  full, full, s16, s16c, fullb],
        out_specs=row,
        out_shape=jax.ShapeDtypeStruct((N, DIM), jnp.float32),
    )(x, h, a8, xr, msg2, msg2, nsum2, nsum2, dcnt2, dcnt2,
      Wsl[:HALF], Wsl[HALF:], Wpt[:HALF], Wpt[HALF:], Wpb[:HALF], Wpb[HALF:],
      Ssrc, Sdst, Sden, Scnt, bias)


def _selectors():
    ssrc = np.zeros((DIM, DIM), np.float32)
    sdst = np.zeros((DIM, DIM), np.float32)
    for hh in range(H):
        ssrc[hh, hh * DH:(hh + 1) * DH] = 1.0
        sdst[4 + hh, hh * DH:(hh + 1) * DH] = 1.0
    sden = np.zeros((16, HALF), np.float32)
    sden[0, 0:DH] = 1.0
    sden[1, DH:2 * DH] = 1.0
    scnt = np.zeros((16, DIM), np.float32)
    scnt[2, :] = 1.0
    sel = np.zeros((NCORES, DIM, 4), np.float32)
    for cdx in range(NCORES):
        sel[cdx, 2 * cdx, 0] = 1.0
        sel[cdx, 2 * cdx + 1, 1] = 1.0
        sel[cdx, 4 + 2 * cdx, 2] = 1.0
        sel[cdx, 5 + 2 * cdx, 3] = 1.0
    return ssrc, sdst, sden, scnt, sel


_SSRC, _SDST, _SDEN, _SCNT, _SEL = _selectors()


def kernel(x, edge_index, W_gat, att_src, att_dst, b_gat, W_sage_l, b_sage_l,
           W_sage_r, W_proj, b_proj, gamma, beta):
    E = edge_index.shape[1]
    E_pad = -(-E // (NTILES * 8 * EB)) * (NTILES * 8 * EB)

    A8 = jnp.zeros((DIM, DIM), jnp.float32)
    for hh in range(H):
        A8 = A8.at[hh * DH:(hh + 1) * DH, hh].set(att_src[hh])
        A8 = A8.at[hh * DH:(hh + 1) * DH, 4 + hh].set(att_dst[hh])

    h, a8, xr, th3, tx3, atab3 = _pre_call(x, W_gat, A8, W_sage_r,
                                           jnp.asarray(_SEL))

    pad = jnp.full((E_pad - E,), N, jnp.int32)
    esrc = jnp.concatenate([edge_index[0].astype(jnp.int32), pad])
    edst = jnp.concatenate([edge_index[1].astype(jnp.int32), pad])
    th = th3.reshape(NCORES * NP, HALF)
    tx = tx3.reshape(NCORES * NP, HALF)
    atab = atab3.reshape(NCORES, 4 * NP)
    z64 = jnp.zeros((EB, HALF), jnp.float32)
    z16 = jnp.zeros((EB, 16), jnp.float32)

    msg2, dcnt2, nsum2 = _sc_call(esrc, edst, th, tx, atab, z64, z16)

    bias = jnp.zeros((8, DIM), jnp.float32)
    bias = bias.at[0].set(b_gat).at[1].set(b_sage_l).at[2].set(b_proj)
    bias = bias.at[3].set(gamma).at[4].set(beta)

    return _post_call(x, h, a8, xr, msg2, nsum2, dcnt2, W_sage_l,
                      W_proj[:DIM], W_proj[DIM:], jnp.asarray(_SSRC),
                      jnp.asarray(_SDST), jnp.asarray(_SDEN),
                      jnp.asarray(_SCNT), bias)
